# baseline (speedup 1.0000x reference)
import functools

import jax
import jax.numpy as jnp
from jax import lax
from jax.experimental import pallas as pl
from jax.experimental.pallas import tpu as pltpu
from jax.experimental.pallas import tpu_sc as plsc

F32 = jnp.float32
N_NODES = 10000
N_PAD = 10240
N_EDGES = 320000
H = 128
CH = 128
N_CHUNKS = N_EDGES // CH
NW = 32
ROWS_PER_TILE = N_PAD // 16
BE = 512
BN = 1000
PI = 3.141592653589793
TWO_PI = 6.283185307179586


def _silu(z):
    return z * (1.0 / (1.0 + jnp.exp(-z)))


def _sc_worker_id():
    return lax.axis_index("s") * 2 + lax.axis_index("c")


def _sc_gather(table, idx, d):
    mesh = plsc.VectorSubcoreMesh(core_axis_name="c", subcore_axis_name="s")
    params = None if d % 128 == 0 else pltpu.CompilerParams(use_tc_tiling_on_sc=False)

    @functools.partial(
        pl.kernel, mesh=mesh,
        out_type=jax.ShapeDtypeStruct((N_EDGES, d), F32),
        compiler_params=params,
        scratch_types=[
            pltpu.VMEM((CH,), jnp.int32),
            pltpu.VMEM((CH, d), F32),
            pltpu.SemaphoreType.DMA,
        ],
    )
    def k(table_hbm, idx_hbm, out_hbm, idx_v, rows_v, sem):
        wid = _sc_worker_id()

        def body(i, carry):
            c = wid + NW * i

            @pl.when(c < N_CHUNKS)
            def _():
                base = c * CH
                pltpu.sync_copy(idx_hbm.at[pl.ds(base, CH)], idx_v)
                pltpu.async_copy(table_hbm.at[idx_v], rows_v, sem).wait()
                pltpu.sync_copy(rows_v, out_hbm.at[pl.ds(base, CH)])
            return carry

        lax.fori_loop(0, (N_CHUNKS + NW - 1) // NW, body, 0)

    return k(table, idx)


def _sc_gather2_add(s_tab, r_tab, send, recv):
    BF = jnp.bfloat16
    mesh = plsc.VectorSubcoreMesh(core_axis_name="c", subcore_axis_name="s")
    params = pltpu.CompilerParams(use_tc_tiling_on_sc=False)

    @functools.partial(
        pl.kernel, mesh=mesh,
        out_type=jax.ShapeDtypeStruct((N_EDGES, H), BF),
        compiler_params=params,
        scratch_types=[
            pltpu.VMEM((2, CH), jnp.int32),
            pltpu.VMEM((2, CH), jnp.int32),
            pltpu.VMEM((CH, H), BF),
            pltpu.VMEM((CH, H), BF),
            pltpu.VMEM((CH, H), BF),
            pltpu.VMEM((CH, H), BF),
            pltpu.SemaphoreType.DMA,
            pltpu.SemaphoreType.DMA,
        ],
    )
    def k(s_hbm, r_hbm, send_hbm, recv_hbm, out_hbm, sidx, ridx,
          rs0, rs1, rr0, rr1, sem0, sem1):
        wid = _sc_worker_id()
        rs = (rs0, rs1)
        rr = (rr0, rr1)
        sems = (sem0, sem1)

        def fetch(i, u):
            c = wid + NW * i

            @pl.when(c < N_CHUNKS)
            def _():
                base = c * CH
                pltpu.sync_copy(send_hbm.at[pl.ds(base, CH)], sidx.at[u])
                pltpu.sync_copy(recv_hbm.at[pl.ds(base, CH)], ridx.at[u])
                pltpu.async_copy(s_hbm.at[sidx.at[u]], rs[u], sems[u])
                pltpu.async_copy(r_hbm.at[ridx.at[u]], rr[u], sems[u])

        def process(i, u):
            c = wid + NW * i

            @pl.when(c < N_CHUNKS)
            def _():
                pltpu.make_async_copy(s_hbm.at[sidx.at[u]], rs[u], sems[u]).wait()
                pltpu.make_async_copy(r_hbm.at[ridx.at[u]], rr[u], sems[u]).wait()

                def addrow(r, cc):
                    for k2 in range(H // 32):
                        sl = pl.ds(k2 * 32, 32)
                        rs[u][r, sl] = rs[u][r, sl] + rr[u][r, sl]
                    return cc

                lax.fori_loop(0, CH, addrow, 0)
                pltpu.sync_copy(rs[u], out_hbm.at[pl.ds(c * CH, CH)])

        fetch(0, 0)

        def body(t, carry):
            i0 = 2 * t
            fetch(i0 + 1, 1)
            process(i0, 0)
            fetch(i0 + 2, 0)
            process(i0 + 1, 1)
            return carry

        n_steps = (N_CHUNKS + NW - 1) // NW
        lax.fori_loop(0, (n_steps + 1) // 2, body, 0)

    return k(s_tab, r_tab, send, recv)


def _sc_gather_pair16(table, send, recv):
    mesh = plsc.VectorSubcoreMesh(core_axis_name="c", subcore_axis_name="s")
    params = pltpu.CompilerParams(use_tc_tiling_on_sc=False)

    @functools.partial(
        pl.kernel, mesh=mesh,
        out_type=[jax.ShapeDtypeStruct((N_EDGES, 16), F32),
                  jax.ShapeDtypeStruct((N_EDGES, 16), F32)],
        compiler_params=params,
        scratch_types=[
            pltpu.VMEM((CH,), jnp.int32),
            pltpu.VMEM((CH,), jnp.int32),
            pltpu.VMEM((CH, 16), F32),
            pltpu.VMEM((CH, 16), F32),
            pltpu.SemaphoreType.DMA,
            pltpu.SemaphoreType.DMA,
        ],
    )
    def k(table_hbm, send_hbm, recv_hbm, ps_hbm, pr_hbm, sidx_v, ridx_v,
          rs_v, rr_v, sem_s, sem_r):
        wid = _sc_worker_id()

        def body(i, carry):
            c = wid + NW * i

            @pl.when(c < N_CHUNKS)
            def _():
                base = c * CH
                pltpu.sync_copy(send_hbm.at[pl.ds(base, CH)], sidx_v)
                pltpu.sync_copy(recv_hbm.at[pl.ds(base, CH)], ridx_v)
                cp_s = pltpu.async_copy(table_hbm.at[sidx_v], rs_v, sem_s)
                cp_r = pltpu.async_copy(table_hbm.at[ridx_v], rr_v, sem_r)
                cp_s.wait()
                cp_r.wait()
                pltpu.sync_copy(rs_v, ps_hbm.at[pl.ds(base, CH)])
                pltpu.sync_copy(rr_v, pr_hbm.at[pl.ds(base, CH)])
            return carry

        lax.fori_loop(0, (N_CHUNKS + NW - 1) // NW, body, 0)

    return k(table, send, recv)


def _sc_scatter_add(m, idx):
    mesh = plsc.VectorSubcoreMesh(core_axis_name="c", subcore_axis_name="s")

    @functools.partial(
        pl.kernel, mesh=mesh,
        out_type=jax.ShapeDtypeStruct((2, N_PAD, H), F32),
        scratch_types=[
            pltpu.VMEM((CH,), jnp.int32),
            pltpu.VMEM((CH, H), F32),
            pltpu.VMEM_SHARED((N_PAD, H), F32),
            pltpu.SemaphoreType.DMA,
        ],
    )
    def k(m_hbm, idx_hbm, out_hbm, idx_v, rows_v, acc_sh, sem):
        cid = lax.axis_index("c")
        sid = lax.axis_index("s")
        wid = sid * 2 + cid

        def zrow(r, carry):
            for k8 in range(H // 16):
                rows_v[r, pl.ds(k8 * 16, 16)] = jnp.zeros((16,), F32)
            return carry

        lax.fori_loop(0, CH, zrow, 0)
        tile_base = sid * ROWS_PER_TILE
        for j in range(ROWS_PER_TILE // CH):
            pltpu.sync_copy(rows_v, acc_sh.at[pl.ds(tile_base + j * CH, CH)])
        plsc.subcore_barrier()

        def body(i, carry):
            c = wid + NW * i

            @pl.when(c < N_CHUNKS)
            def _():
                base = c * CH
                pltpu.sync_copy(idx_hbm.at[pl.ds(base, CH)], idx_v)
                pltpu.sync_copy(m_hbm.at[pl.ds(base, CH)], rows_v)
                pltpu.sync_copy(rows_v, acc_sh.at[idx_v], add=True)
            return carry

        lax.fori_loop(0, (N_CHUNKS + NW - 1) // NW, body, 0)
        plsc.subcore_barrier()

        for j in range(ROWS_PER_TILE // CH):
            base = tile_base + j * CH
            pltpu.sync_copy(acc_sh.at[pl.ds(base, CH)], rows_v)
            pltpu.sync_copy(rows_v, out_hbm.at[cid, pl.ds(base, CH)])

    return k(m, idx)


def _sc_count(idx):
    mesh = plsc.VectorSubcoreMesh(core_axis_name="c", subcore_axis_name="s")

    @functools.partial(
        pl.kernel, mesh=mesh,
        out_type=jax.ShapeDtypeStruct((2, N_PAD), F32),
        scratch_types=[
            pltpu.VMEM((CH,), jnp.int32),
            pltpu.VMEM((CH,), F32),
            pltpu.VMEM((CH,), F32),
            pltpu.VMEM_SHARED((N_PAD,), F32),
            pltpu.SemaphoreType.DMA,
        ],
    )
    def k(idx_hbm, out_hbm, idx_v, ones_v, zeros_v, acc_sh, sem):
        cid = lax.axis_index("c")
        sid = lax.axis_index("s")
        wid = sid * 2 + cid

        for k8 in range(CH // 16):
            ones_v[pl.ds(k8 * 16, 16)] = jnp.full((16,), 1.0, F32)
            zeros_v[pl.ds(k8 * 16, 16)] = jnp.zeros((16,), F32)
        tile_base = sid * ROWS_PER_TILE
        for j in range(ROWS_PER_TILE // CH):
            pltpu.sync_copy(zeros_v, acc_sh.at[pl.ds(tile_base + j * CH, CH)])
        plsc.subcore_barrier()

        def body(i, carry):
            c = wid + NW * i

            @pl.when(c < N_CHUNKS)
            def _():
                base = c * CH
                pltpu.sync_copy(idx_hbm.at[pl.ds(base, CH)], idx_v)
                pltpu.sync_copy(ones_v, acc_sh.at[idx_v], add=True)
            return carry

        lax.fori_loop(0, (N_CHUNKS + NW - 1) // NW, body, 0)
        plsc.subcore_barrier()

        for j in range(ROWS_PER_TILE // CH):
            base = tile_base + j * CH
            pltpu.sync_copy(acc_sh.at[pl.ds(base, CH)], zeros_v)
            pltpu.sync_copy(zeros_v, out_hbm.at[cid, pl.ds(base, CH)])

    return k(idx)


def _prep_body(x_ref, vel_ref, wrow_ref, bias_ref, tab_ref, res_ref):
    xx = x_ref[...]
    vv = vel_ref[...]
    vx = vv[:, 0:1]
    vy = vv[:, 1:2]
    theta = jnp.arctan2(vy, vx)
    c = jnp.cos(theta)
    s = jnp.sin(theta)
    speed = jnp.sqrt(vx * vx + vy * vy)
    z = jnp.zeros_like(vx)
    tab_ref[...] = jnp.concatenate(
        [xx[:, 0:1], xx[:, 1:2], vx, vy, theta, c, s, speed,
         z, z, z, z, z, z, z, z], axis=1)
    res_ref[...] = speed * wrow_ref[...] + bias_ref[...]


def _tc_prep(x, vel, res_row, res_bias):
    grid = N_NODES // BN
    return pl.pallas_call(
        _prep_body,
        grid=(grid,),
        in_specs=[
            pl.BlockSpec((BN, 2), lambda i: (i, 0)),
            pl.BlockSpec((BN, 2), lambda i: (i, 0)),
            pl.BlockSpec((1, H), lambda i: (0, 0)),
            pl.BlockSpec((1, H), lambda i: (0, 0)),
        ],
        out_specs=[
            pl.BlockSpec((BN, 16), lambda i: (i, 0)),
            pl.BlockSpec((BN, H), lambda i: (i, 0)),
        ],
        out_shape=[
            jax.ShapeDtypeStruct((N_NODES, 16), F32),
            jax.ShapeDtypeStruct((N_NODES, H), F32),
        ],
    )(x, vel, res_row, res_bias)


def _edge1_body(ps_ref, pr_ref, ea_ref, w1_ref, b1_ref, w2_ref, b2_ref, m_ref):
    Ps = ps_ref[...]
    Pr = pr_ref[...]
    EA = ea_ref[...]
    dx = Ps[:, 0:1] - Pr[:, 0:1]
    dy = Ps[:, 1:2] - Pr[:, 1:2]
    cr = Pr[:, 5:6]
    sr = Pr[:, 6:7]
    rrx = cr * dx + sr * dy
    rry = -sr * dx + cr * dy
    d = Ps[:, 4:5] - Pr[:, 4:5]
    reul = d - jnp.where(d > PI, TWO_PI, 0.0) + jnp.where(d < -PI, TWO_PI, 0.0)
    dist = jnp.sqrt(dx * dx + dy * dy)
    sph = jnp.arctan2(rry, rrx)
    vxs = Ps[:, 2:3]
    vys = Ps[:, 3:4]
    rvx = cr * vxs + sr * vys
    rvy = -sr * vxs + cr * vys
    spr = Pr[:, 7:8]
    z = jnp.zeros_like(dx)
    feat = jnp.concatenate(
        [rrx, rry, reul, dist, sph, rvx, rvy, z, z, spr, z,
         EA[:, 0:1], EA[:, 1:2], z, z, z], axis=1)
    m1 = _silu(jnp.dot(feat, w1_ref[...], preferred_element_type=F32) + b1_ref[...])
    m_ref[...] = _silu(jnp.dot(m1, w2_ref[...], preferred_element_type=F32) + b2_ref[...])


def _tc_edge1(ps, pr, ea, w1p, b1, w2, b2):
    grid = N_EDGES // BE
    return pl.pallas_call(
        _edge1_body,
        grid=(grid,),
        in_specs=[
            pl.BlockSpec((BE, 16), lambda i: (i, 0)),
            pl.BlockSpec((BE, 16), lambda i: (i, 0)),
            pl.BlockSpec((BE, 2), lambda i: (i, 0)),
            pl.BlockSpec((16, H), lambda i: (0, 0)),
            pl.BlockSpec((1, H), lambda i: (0, 0)),
            pl.BlockSpec((H, H), lambda i: (0, 0)),
            pl.BlockSpec((1, H), lambda i: (0, 0)),
        ],
        out_specs=pl.BlockSpec((BE, H), lambda i: (i, 0)),
        out_shape=jax.ShapeDtypeStruct((N_EDGES, H), F32),
    )(ps, pr, ea, w1p, b1, w2, b2)


def _edgeN_body(mp_ref, g_ref, w1_ref, b1_ref, w2_ref, b2_ref, m_ref):
    pre = (jnp.dot(mp_ref[...], w1_ref[...], preferred_element_type=F32)
           + g_ref[...].astype(F32) + b1_ref[...])
    m1 = _silu(pre)
    m_ref[...] = _silu(jnp.dot(m1, w2_ref[...], preferred_element_type=F32) + b2_ref[...])


def _tc_edgeN(m_prev, g, w1e, b1, w2, b2):
    grid = N_EDGES // BE
    return pl.pallas_call(
        _edgeN_body,
        grid=(grid,),
        in_specs=[
            pl.BlockSpec((BE, H), lambda i: (i, 0)),
            pl.BlockSpec((BE, H), lambda i: (i, 0)),
            pl.BlockSpec((H, H), lambda i: (0, 0)),
            pl.BlockSpec((1, H), lambda i: (0, 0)),
            pl.BlockSpec((H, H), lambda i: (0, 0)),
            pl.BlockSpec((1, H), lambda i: (0, 0)),
        ],
        out_specs=pl.BlockSpec((BE, H), lambda i: (i, 0)),
        out_shape=jax.ShapeDtypeStruct((N_EDGES, H), F32),
    )(m_prev, g, w1e, b1, w2, b2)


def _node_body(res_ref, parts_ref, rdeg_ref, uw1_ref, ub1_ref, uw2_ref, ub2_ref,
               ws_ref, wr_ref, xn_ref, s_ref, r_ref):
    aggr = (parts_ref[0] + parts_ref[1]) * rdeg_ref[...]
    xn1 = res_ref[...] + aggr
    u = _silu(jnp.dot(xn1, uw1_ref[...], preferred_element_type=F32) + ub1_ref[...])
    u = jnp.dot(u, uw2_ref[...], preferred_element_type=F32) + ub2_ref[...]
    xn = xn1 + u
    xn_ref[...] = xn
    s_ref[...] = jnp.dot(xn, ws_ref[...], preferred_element_type=F32).astype(jnp.bfloat16)
    r_ref[...] = jnp.dot(xn, wr_ref[...], preferred_element_type=F32).astype(jnp.bfloat16)


def _tc_node(res, parts, rdeg, uw1, ub1, uw2, ub2, ws, wr):
    grid = N_NODES // BN
    return pl.pallas_call(
        _node_body,
        grid=(grid,),
        in_specs=[
            pl.BlockSpec((BN, H), lambda i: (i, 0)),
            pl.BlockSpec((2, BN, H), lambda i: (0, i, 0)),
            pl.BlockSpec((BN, 1), lambda i: (i, 0)),
            pl.BlockSpec((H, 2 * H), lambda i: (0, 0)),
            pl.BlockSpec((1, 2 * H), lambda i: (0, 0)),
            pl.BlockSpec((2 * H, H), lambda i: (0, 0)),
            pl.BlockSpec((1, H), lambda i: (0, 0)),
            pl.BlockSpec((H, H), lambda i: (0, 0)),
            pl.BlockSpec((H, H), lambda i: (0, 0)),
        ],
        out_specs=[
            pl.BlockSpec((BN, H), lambda i: (i, 0)),
            pl.BlockSpec((BN, H), lambda i: (i, 0)),
            pl.BlockSpec((BN, H), lambda i: (i, 0)),
        ],
        out_shape=[
            jax.ShapeDtypeStruct((N_NODES, H), F32),
            jax.ShapeDtypeStruct((N_NODES, H), jnp.bfloat16),
            jax.ShapeDtypeStruct((N_NODES, H), jnp.bfloat16),
        ],
    )(res, parts, rdeg, uw1, ub1, uw2, ub2, ws, wr)


def _final_body(res_ref, parts_ref, rdeg_ref, uw1_ref, ub1_ref, uw2_ref, ub2_ref,
                ow1_ref, ob1_ref, ow2_ref, ob2_ref, ow3_ref, ob3_ref,
                x_ref, tab_ref, out_ref):
    aggr = (parts_ref[0] + parts_ref[1]) * rdeg_ref[...]
    xn1 = res_ref[...] + aggr
    u = _silu(jnp.dot(xn1, uw1_ref[...], preferred_element_type=F32) + ub1_ref[...])
    u = jnp.dot(u, uw2_ref[...], preferred_element_type=F32) + ub2_ref[...]
    xn = xn1 + u
    o = _silu(jnp.dot(xn, ow1_ref[...], preferred_element_type=F32) + ob1_ref[...])
    o = _silu(jnp.dot(o, ow2_ref[...], preferred_element_type=F32) + ob2_ref[...])
    pred = jnp.dot(o, ow3_ref[...], preferred_element_type=F32) + ob3_ref[...]
    p0 = pred[:, 0:1]
    p1 = pred[:, 1:2]
    c = tab_ref[:, 5:6]
    s = tab_ref[:, 6:7]
    out_ref[...] = x_ref[...] + jnp.concatenate(
        [c * p0 - s * p1, s * p0 + c * p1], axis=1)


def _tc_final(res, parts, rdeg, uw1, ub1, uw2, ub2,
              ow1, ob1, ow2, ob2, ow3p, ob3p, x, tab):
    grid = N_NODES // BN
    return pl.pallas_call(
        _final_body,
        grid=(grid,),
        in_specs=[
            pl.BlockSpec((BN, H), lambda i: (i, 0)),
            pl.BlockSpec((2, BN, H), lambda i: (0, i, 0)),
            pl.BlockSpec((BN, 1), lambda i: (i, 0)),
            pl.BlockSpec((H, 2 * H), lambda i: (0, 0)),
            pl.BlockSpec((1, 2 * H), lambda i: (0, 0)),
            pl.BlockSpec((2 * H, H), lambda i: (0, 0)),
            pl.BlockSpec((1, H), lambda i: (0, 0)),
            pl.BlockSpec((H, H), lambda i: (0, 0)),
            pl.BlockSpec((1, H), lambda i: (0, 0)),
            pl.BlockSpec((H, H), lambda i: (0, 0)),
            pl.BlockSpec((1, H), lambda i: (0, 0)),
            pl.BlockSpec((H, H), lambda i: (0, 0)),
            pl.BlockSpec((1, H), lambda i: (0, 0)),
            pl.BlockSpec((BN, 2), lambda i: (i, 0)),
            pl.BlockSpec((BN, 16), lambda i: (i, 0)),
        ],
        out_specs=pl.BlockSpec((BN, 2), lambda i: (i, 0)),
        out_shape=jax.ShapeDtypeStruct((N_NODES, 2), F32),
    )(res, parts, rdeg, uw1, ub1, uw2, ub2, ow1, ob1, ow2, ob2, ow3p, ob3p, x, tab)


def kernel(h, x, vel, edges, edge_attr_orig,
           msg_W1_1, msg_b1_1, msg_W1_2, msg_b1_2, msg_W1_3, msg_b1_3,
           msg_W1_4, msg_b1_4,
           msg_W2_1, msg_b2_1, msg_W2_2, msg_b2_2, msg_W2_3, msg_b2_3,
           msg_W2_4, msg_b2_4,
           upd_W1_1, upd_b1_1, upd_W1_2, upd_b1_2, upd_W1_3, upd_b1_3,
           upd_W1_4, upd_b1_4,
           upd_W2_1, upd_b2_1, upd_W2_2, upd_b2_2, upd_W2_3, upd_b2_3,
           upd_W2_4, upd_b2_4,
           res_W_1, res_b_1, out_W1, out_b1, out_W2, out_b2, out_W3, out_b3):
    del h
    send = edges[0]
    recv = edges[1]

    msg_w1 = {2: msg_W1_2, 3: msg_W1_3, 4: msg_W1_4}
    msg_b1 = {1: msg_b1_1.reshape(1, H), 2: msg_b1_2.reshape(1, H),
              3: msg_b1_3.reshape(1, H), 4: msg_b1_4.reshape(1, H)}
    msg_w2 = {1: msg_W2_1, 2: msg_W2_2, 3: msg_W2_3, 4: msg_W2_4}
    msg_b2 = {1: msg_b2_1.reshape(1, H), 2: msg_b2_2.reshape(1, H),
              3: msg_b2_3.reshape(1, H), 4: msg_b2_4.reshape(1, H)}
    upd_w1 = {1: upd_W1_1, 2: upd_W1_2, 3: upd_W1_3, 4: upd_W1_4}
    upd_b1 = {i: b.reshape(1, 2 * H) for i, b in
              {1: upd_b1_1, 2: upd_b1_2, 3: upd_b1_3, 4: upd_b1_4}.items()}
    upd_w2 = {1: upd_W2_1, 2: upd_W2_2, 3: upd_W2_3, 4: upd_W2_4}
    upd_b2 = {i: b.reshape(1, H) for i, b in
              {1: upd_b2_1, 2: upd_b2_2, 3: upd_b2_3, 4: upd_b2_4}.items()}
    w1s = {i: msg_w1[i][0:H] for i in (2, 3, 4)}
    w1r = {i: msg_w1[i][H:2 * H] for i in (2, 3, 4)}
    w1e = {i: msg_w1[i][2 * H:3 * H] for i in (2, 3, 4)}

    w1_1p = jnp.concatenate([msg_W1_1, jnp.zeros((3, H), F32)], axis=0)
    ow3p = jnp.concatenate([out_W3, jnp.zeros((H, H - 2), F32)], axis=1)
    ob3p = jnp.concatenate([out_b3, jnp.zeros((H - 2,), F32)]).reshape(1, H)

    tab, res1 = _tc_prep(x, vel, res_W_1[2:3, :], res_b_1.reshape(1, H))

    cnt = _sc_count(recv)
    rdeg = (1.0 / jnp.maximum(cnt[0] + cnt[1], 1.0)).reshape(N_PAD, 1)

    ps, pr = _sc_gather_pair16(tab, send, recv)
    m = _tc_edge1(ps, pr, edge_attr_orig, w1_1p, msg_b1[1], msg_w2[1], msg_b2[1])

    parts = _sc_scatter_add(m, recv)
    res = res1
    for i in (2, 3, 4):
        xn, s_tab, r_tab = _tc_node(res, parts, rdeg,
                                    upd_w1[i - 1], upd_b1[i - 1],
                                    upd_w2[i - 1], upd_b2[i - 1],
                                    w1s[i], w1r[i])
        g = _sc_gather2_add(s_tab, r_tab, send, recv)
        m = _tc_edgeN(m, g, w1e[i], msg_b1[i], msg_w2[i], msg_b2[i])
        parts = _sc_scatter_add(m, recv)
        res = xn

    return _tc_final(res, parts, rdeg,
                     upd_w1[4], upd_b1[4], upd_w2[4], upd_b2[4],
                     out_W1, out_b1.reshape(1, H), out_W2, out_b2.reshape(1, H),
                     ow3p, ob3p, x, tab)


# speedup vs baseline: 1.3151x; 1.3151x over previous
import functools

import jax
import jax.numpy as jnp
from jax import lax
from jax.experimental import pallas as pl
from jax.experimental.pallas import tpu as pltpu
from jax.experimental.pallas import tpu_sc as plsc

F32 = jnp.float32
N_NODES = 10000
N_PAD = 10240
N_EDGES = 320000
H = 128
CH = 128
N_CHUNKS = N_EDGES // CH
NW = 32
ROWS_PER_TILE = N_PAD // 16
BE = 512
BN = 1000
PI = 3.141592653589793
TWO_PI = 6.283185307179586


def _silu(z):
    return z * (1.0 / (1.0 + jnp.exp(-z)))


def _sc_worker_id():
    return lax.axis_index("s") * 2 + lax.axis_index("c")


def _sc_gather(table, idx, d):
    mesh = plsc.VectorSubcoreMesh(core_axis_name="c", subcore_axis_name="s")
    params = None if d % 128 == 0 else pltpu.CompilerParams(use_tc_tiling_on_sc=False)

    @functools.partial(
        pl.kernel, mesh=mesh,
        out_type=jax.ShapeDtypeStruct((N_EDGES, d), F32),
        compiler_params=params,
        scratch_types=[
            pltpu.VMEM((CH,), jnp.int32),
            pltpu.VMEM((CH, d), F32),
            pltpu.SemaphoreType.DMA,
        ],
    )
    def k(table_hbm, idx_hbm, out_hbm, idx_v, rows_v, sem):
        wid = _sc_worker_id()

        def body(i, carry):
            c = wid + NW * i

            @pl.when(c < N_CHUNKS)
            def _():
                base = c * CH
                pltpu.sync_copy(idx_hbm.at[pl.ds(base, CH)], idx_v)
                pltpu.async_copy(table_hbm.at[idx_v], rows_v, sem).wait()
                pltpu.sync_copy(rows_v, out_hbm.at[pl.ds(base, CH)])
            return carry

        lax.fori_loop(0, (N_CHUNKS + NW - 1) // NW, body, 0)

    return k(table, idx)


def _sc_gather2_add(s_tab, r_tab, send, recv):
    mesh = plsc.VectorSubcoreMesh(core_axis_name="c", subcore_axis_name="s")

    @functools.partial(
        pl.kernel, mesh=mesh,
        out_type=jax.ShapeDtypeStruct((N_EDGES, H), F32),
        scratch_types=[
            pltpu.VMEM((2, CH), jnp.int32),
            pltpu.VMEM((2, CH), jnp.int32),
            pltpu.VMEM((CH, H), F32),
            pltpu.VMEM((CH, H), F32),
            pltpu.VMEM((CH, H), F32),
            pltpu.VMEM((CH, H), F32),
            pltpu.SemaphoreType.DMA,
            pltpu.SemaphoreType.DMA,
        ],
    )
    def k(s_hbm, r_hbm, send_hbm, recv_hbm, out_hbm, sidx, ridx,
          rs0, rs1, rr0, rr1, sem0, sem1):
        wid = _sc_worker_id()
        rs = (rs0, rs1)
        rr = (rr0, rr1)
        sems = (sem0, sem1)

        def fetch(i, u):
            c = wid + NW * i

            @pl.when(c < N_CHUNKS)
            def _():
                base = c * CH
                pltpu.sync_copy(send_hbm.at[pl.ds(base, CH)], sidx.at[u])
                pltpu.sync_copy(recv_hbm.at[pl.ds(base, CH)], ridx.at[u])
                pltpu.async_copy(s_hbm.at[sidx.at[u]], rs[u], sems[u])
                pltpu.async_copy(r_hbm.at[ridx.at[u]], rr[u], sems[u])

        def process(i, u):
            c = wid + NW * i

            @pl.when(c < N_CHUNKS)
            def _():
                pltpu.make_async_copy(s_hbm.at[sidx.at[u]], rs[u], sems[u]).wait()
                pltpu.make_async_copy(r_hbm.at[ridx.at[u]], rr[u], sems[u]).wait()

                def addrow(r, cc):
                    for k2 in range(H // 16):
                        sl = pl.ds(k2 * 16, 16)
                        rs[u][r, sl] = rs[u][r, sl] + rr[u][r, sl]
                    return cc

                lax.fori_loop(0, CH, addrow, 0)
                pltpu.sync_copy(rs[u], out_hbm.at[pl.ds(c * CH, CH)])

        fetch(0, 0)

        def body(t, carry):
            i0 = 2 * t
            fetch(i0 + 1, 1)
            process(i0, 0)
            fetch(i0 + 2, 0)
            process(i0 + 1, 1)
            return carry

        n_steps = (N_CHUNKS + NW - 1) // NW
        lax.fori_loop(0, (n_steps + 1) // 2, body, 0)

    return k(s_tab, r_tab, send, recv)


def _sc_gather_pair16(table, send, recv):
    mesh = plsc.VectorSubcoreMesh(core_axis_name="c", subcore_axis_name="s")
    params = pltpu.CompilerParams(use_tc_tiling_on_sc=False)

    @functools.partial(
        pl.kernel, mesh=mesh,
        out_type=[jax.ShapeDtypeStruct((N_EDGES, 16), F32),
                  jax.ShapeDtypeStruct((N_EDGES, 16), F32)],
        compiler_params=params,
        scratch_types=[
            pltpu.VMEM((CH,), jnp.int32),
            pltpu.VMEM((CH,), jnp.int32),
            pltpu.VMEM((CH, 16), F32),
            pltpu.VMEM((CH, 16), F32),
            pltpu.SemaphoreType.DMA,
            pltpu.SemaphoreType.DMA,
        ],
    )
    def k(table_hbm, send_hbm, recv_hbm, ps_hbm, pr_hbm, sidx_v, ridx_v,
          rs_v, rr_v, sem_s, sem_r):
        wid = _sc_worker_id()

        def body(i, carry):
            c = wid + NW * i

            @pl.when(c < N_CHUNKS)
            def _():
                base = c * CH
                pltpu.sync_copy(send_hbm.at[pl.ds(base, CH)], sidx_v)
                pltpu.sync_copy(recv_hbm.at[pl.ds(base, CH)], ridx_v)
                cp_s = pltpu.async_copy(table_hbm.at[sidx_v], rs_v, sem_s)
                cp_r = pltpu.async_copy(table_hbm.at[ridx_v], rr_v, sem_r)
                cp_s.wait()
                cp_r.wait()
                pltpu.sync_copy(rs_v, ps_hbm.at[pl.ds(base, CH)])
                pltpu.sync_copy(rr_v, pr_hbm.at[pl.ds(base, CH)])
            return carry

        lax.fori_loop(0, (N_CHUNKS + NW - 1) // NW, body, 0)

    return k(table, send, recv)


def _sc_scatter_add(m, idx):
    mesh = plsc.VectorSubcoreMesh(core_axis_name="c", subcore_axis_name="s")

    @functools.partial(
        pl.kernel, mesh=mesh,
        out_type=jax.ShapeDtypeStruct((2, N_PAD, H), F32),
        scratch_types=[
            pltpu.VMEM((CH,), jnp.int32),
            pltpu.VMEM((CH, H), F32),
            pltpu.VMEM_SHARED((N_PAD, H), F32),
            pltpu.SemaphoreType.DMA,
        ],
    )
    def k(m_hbm, idx_hbm, out_hbm, idx_v, rows_v, acc_sh, sem):
        cid = lax.axis_index("c")
        sid = lax.axis_index("s")
        wid = sid * 2 + cid

        def zrow(r, carry):
            for k8 in range(H // 16):
                rows_v[r, pl.ds(k8 * 16, 16)] = jnp.zeros((16,), F32)
            return carry

        lax.fori_loop(0, CH, zrow, 0)
        tile_base = sid * ROWS_PER_TILE
        for j in range(ROWS_PER_TILE // CH):
            pltpu.sync_copy(rows_v, acc_sh.at[pl.ds(tile_base + j * CH, CH)])
        plsc.subcore_barrier()

        def body(i, carry):
            c = wid + NW * i

            @pl.when(c < N_CHUNKS)
            def _():
                base = c * CH
                pltpu.sync_copy(idx_hbm.at[pl.ds(base, CH)], idx_v)
                pltpu.sync_copy(m_hbm.at[pl.ds(base, CH)], rows_v)
                pltpu.sync_copy(rows_v, acc_sh.at[idx_v], add=True)
            return carry

        lax.fori_loop(0, (N_CHUNKS + NW - 1) // NW, body, 0)
        plsc.subcore_barrier()

        for j in range(ROWS_PER_TILE // CH):
            base = tile_base + j * CH
            pltpu.sync_copy(acc_sh.at[pl.ds(base, CH)], rows_v)
            pltpu.sync_copy(rows_v, out_hbm.at[cid, pl.ds(base, CH)])

    return k(m, idx)


def _sc_count(idx):
    mesh = plsc.VectorSubcoreMesh(core_axis_name="c", subcore_axis_name="s")

    @functools.partial(
        pl.kernel, mesh=mesh,
        out_type=jax.ShapeDtypeStruct((2, N_PAD), F32),
        scratch_types=[
            pltpu.VMEM((CH,), jnp.int32),
            pltpu.VMEM((CH,), F32),
            pltpu.VMEM((CH,), F32),
            pltpu.VMEM_SHARED((N_PAD,), F32),
            pltpu.SemaphoreType.DMA,
        ],
    )
    def k(idx_hbm, out_hbm, idx_v, ones_v, zeros_v, acc_sh, sem):
        cid = lax.axis_index("c")
        sid = lax.axis_index("s")
        wid = sid * 2 + cid

        for k8 in range(CH // 16):
            ones_v[pl.ds(k8 * 16, 16)] = jnp.full((16,), 1.0, F32)
            zeros_v[pl.ds(k8 * 16, 16)] = jnp.zeros((16,), F32)
        tile_base = sid * ROWS_PER_TILE
        for j in range(ROWS_PER_TILE // CH):
            pltpu.sync_copy(zeros_v, acc_sh.at[pl.ds(tile_base + j * CH, CH)])
        plsc.subcore_barrier()

        def body(i, carry):
            c = wid + NW * i

            @pl.when(c < N_CHUNKS)
            def _():
                base = c * CH
                pltpu.sync_copy(idx_hbm.at[pl.ds(base, CH)], idx_v)
                pltpu.sync_copy(ones_v, acc_sh.at[idx_v], add=True)
            return carry

        lax.fori_loop(0, (N_CHUNKS + NW - 1) // NW, body, 0)
        plsc.subcore_barrier()

        for j in range(ROWS_PER_TILE // CH):
            base = tile_base + j * CH
            pltpu.sync_copy(acc_sh.at[pl.ds(base, CH)], zeros_v)
            pltpu.sync_copy(zeros_v, out_hbm.at[cid, pl.ds(base, CH)])

    return k(idx)


def _prep_body(x_ref, vel_ref, wrow_ref, bias_ref, tab_ref, res_ref):
    xx = x_ref[...]
    vv = vel_ref[...]
    vx = vv[:, 0:1]
    vy = vv[:, 1:2]
    theta = jnp.arctan2(vy, vx)
    c = jnp.cos(theta)
    s = jnp.sin(theta)
    speed = jnp.sqrt(vx * vx + vy * vy)
    z = jnp.zeros_like(vx)
    tab_ref[...] = jnp.concatenate(
        [xx[:, 0:1], xx[:, 1:2], vx, vy, theta, c, s, speed,
         z, z, z, z, z, z, z, z], axis=1)
    res_ref[...] = speed * wrow_ref[...] + bias_ref[...]


def _tc_prep(x, vel, res_row, res_bias):
    grid = N_NODES // BN
    return pl.pallas_call(
        _prep_body,
        grid=(grid,),
        in_specs=[
            pl.BlockSpec((BN, 2), lambda i: (i, 0)),
            pl.BlockSpec((BN, 2), lambda i: (i, 0)),
            pl.BlockSpec((1, H), lambda i: (0, 0)),
            pl.BlockSpec((1, H), lambda i: (0, 0)),
        ],
        out_specs=[
            pl.BlockSpec((BN, 16), lambda i: (i, 0)),
            pl.BlockSpec((BN, H), lambda i: (i, 0)),
        ],
        out_shape=[
            jax.ShapeDtypeStruct((N_NODES, 16), F32),
            jax.ShapeDtypeStruct((N_NODES, H), F32),
        ],
    )(x, vel, res_row, res_bias)


def _edge1_body(ps_ref, pr_ref, ea_ref, w1_ref, b1_ref, w2_ref, b2_ref, m_ref):
    PsT = ps_ref[...].T
    PrT = pr_ref[...].T
    EAT = ea_ref[...].T

    def row(M, r):
        return M[r:r + 1, :]

    dx = row(PsT, 0) - row(PrT, 0)
    dy = row(PsT, 1) - row(PrT, 1)
    cr = row(PrT, 5)
    sr = row(PrT, 6)
    rrx = cr * dx + sr * dy
    rry = -sr * dx + cr * dy
    d = row(PsT, 4) - row(PrT, 4)
    reul = d - jnp.where(d > PI, TWO_PI, 0.0) + jnp.where(d < -PI, TWO_PI, 0.0)
    dist = jnp.sqrt(dx * dx + dy * dy)
    sph = jnp.arctan2(rry, rrx)
    vxs = row(PsT, 2)
    vys = row(PsT, 3)
    rvx = cr * vxs + sr * vys
    rvy = -sr * vxs + cr * vys
    spr = row(PrT, 7)
    z = jnp.zeros_like(dx)
    featT = jnp.concatenate(
        [rrx, rry, reul, dist, sph, rvx, rvy, z, z, spr, z,
         row(EAT, 0), row(EAT, 1), z, z, z], axis=0)
    feat = featT.T.astype(jnp.bfloat16)
    m1 = _silu(jnp.dot(feat, w1_ref[...], preferred_element_type=F32) + b1_ref[...])
    m_ref[...] = _silu(jnp.dot(m1.astype(jnp.bfloat16), w2_ref[...],
                               preferred_element_type=F32) + b2_ref[...])


def _tc_edge1(ps, pr, ea, w1p, b1, w2, b2):
    grid = N_EDGES // BE
    return pl.pallas_call(
        _edge1_body,
        grid=(grid,),
        in_specs=[
            pl.BlockSpec((BE, 16), lambda i: (i, 0)),
            pl.BlockSpec((BE, 16), lambda i: (i, 0)),
            pl.BlockSpec((BE, 2), lambda i: (i, 0)),
            pl.BlockSpec((16, H), lambda i: (0, 0)),
            pl.BlockSpec((1, H), lambda i: (0, 0)),
            pl.BlockSpec((H, H), lambda i: (0, 0)),
            pl.BlockSpec((1, H), lambda i: (0, 0)),
        ],
        out_specs=pl.BlockSpec((BE, H), lambda i: (i, 0)),
        out_shape=jax.ShapeDtypeStruct((N_EDGES, H), F32),
    )(ps, pr, ea, w1p, b1, w2, b2)


def _edgeN_body(mp_ref, g_ref, w1_ref, b1_ref, w2_ref, b2_ref, m_ref):
    pre = (jnp.dot(mp_ref[...].astype(jnp.bfloat16), w1_ref[...],
                   preferred_element_type=F32)
           + g_ref[...] + b1_ref[...])
    m1 = _silu(pre)
    m_ref[...] = _silu(jnp.dot(m1.astype(jnp.bfloat16), w2_ref[...],
                               preferred_element_type=F32) + b2_ref[...])


def _tc_edgeN(m_prev, g, w1e, b1, w2, b2):
    grid = N_EDGES // BE
    return pl.pallas_call(
        _edgeN_body,
        grid=(grid,),
        in_specs=[
            pl.BlockSpec((BE, H), lambda i: (i, 0)),
            pl.BlockSpec((BE, H), lambda i: (i, 0)),
            pl.BlockSpec((H, H), lambda i: (0, 0)),
            pl.BlockSpec((1, H), lambda i: (0, 0)),
            pl.BlockSpec((H, H), lambda i: (0, 0)),
            pl.BlockSpec((1, H), lambda i: (0, 0)),
        ],
        out_specs=pl.BlockSpec((BE, H), lambda i: (i, 0)),
        out_shape=jax.ShapeDtypeStruct((N_EDGES, H), F32),
    )(m_prev, g, w1e, b1, w2, b2)


def _node_body(res_ref, parts_ref, rdeg_ref, uw1_ref, ub1_ref, uw2_ref, ub2_ref,
               ws_ref, wr_ref, xn_ref, s_ref, r_ref):
    aggr = (parts_ref[0] + parts_ref[1]) * rdeg_ref[...]
    xn1 = res_ref[...] + aggr
    u = _silu(jnp.dot(xn1, uw1_ref[...], preferred_element_type=F32) + ub1_ref[...])
    u = jnp.dot(u, uw2_ref[...], preferred_element_type=F32) + ub2_ref[...]
    xn = xn1 + u
    xn_ref[...] = xn
    s_ref[...] = jnp.dot(xn, ws_ref[...], preferred_element_type=F32)
    r_ref[...] = jnp.dot(xn, wr_ref[...], preferred_element_type=F32)


def _tc_node(res, parts, rdeg, uw1, ub1, uw2, ub2, ws, wr):
    grid = N_NODES // BN
    return pl.pallas_call(
        _node_body,
        grid=(grid,),
        in_specs=[
            pl.BlockSpec((BN, H), lambda i: (i, 0)),
            pl.BlockSpec((2, BN, H), lambda i: (0, i, 0)),
            pl.BlockSpec((BN, 1), lambda i: (i, 0)),
            pl.BlockSpec((H, 2 * H), lambda i: (0, 0)),
            pl.BlockSpec((1, 2 * H), lambda i: (0, 0)),
            pl.BlockSpec((2 * H, H), lambda i: (0, 0)),
            pl.BlockSpec((1, H), lambda i: (0, 0)),
            pl.BlockSpec((H, H), lambda i: (0, 0)),
            pl.BlockSpec((H, H), lambda i: (0, 0)),
        ],
        out_specs=[
            pl.BlockSpec((BN, H), lambda i: (i, 0)),
            pl.BlockSpec((BN, H), lambda i: (i, 0)),
            pl.BlockSpec((BN, H), lambda i: (i, 0)),
        ],
        out_shape=[
            jax.ShapeDtypeStruct((N_NODES, H), F32),
            jax.ShapeDtypeStruct((N_NODES, H), F32),
            jax.ShapeDtypeStruct((N_NODES, H), F32),
        ],
    )(res, parts, rdeg, uw1, ub1, uw2, ub2, ws, wr)


def _final_body(res_ref, parts_ref, rdeg_ref, uw1_ref, ub1_ref, uw2_ref, ub2_ref,
                ow1_ref, ob1_ref, ow2_ref, ob2_ref, ow3_ref, ob3_ref,
                x_ref, tab_ref, out_ref):
    aggr = (parts_ref[0] + parts_ref[1]) * rdeg_ref[...]
    xn1 = res_ref[...] + aggr
    u = _silu(jnp.dot(xn1, uw1_ref[...], preferred_element_type=F32) + ub1_ref[...])
    u = jnp.dot(u, uw2_ref[...], preferred_element_type=F32) + ub2_ref[...]
    xn = xn1 + u
    o = _silu(jnp.dot(xn, ow1_ref[...], preferred_element_type=F32) + ob1_ref[...])
    o = _silu(jnp.dot(o, ow2_ref[...], preferred_element_type=F32) + ob2_ref[...])
    pred = jnp.dot(o, ow3_ref[...], preferred_element_type=F32) + ob3_ref[...]
    p0 = pred[:, 0:1]
    p1 = pred[:, 1:2]
    c = tab_ref[:, 5:6]
    s = tab_ref[:, 6:7]
    out_ref[...] = x_ref[...] + jnp.concatenate(
        [c * p0 - s * p1, s * p0 + c * p1], axis=1)


def _tc_final(res, parts, rdeg, uw1, ub1, uw2, ub2,
              ow1, ob1, ow2, ob2, ow3p, ob3p, x, tab):
    grid = N_NODES // BN
    return pl.pallas_call(
        _final_body,
        grid=(grid,),
        in_specs=[
            pl.BlockSpec((BN, H), lambda i: (i, 0)),
            pl.BlockSpec((2, BN, H), lambda i: (0, i, 0)),
            pl.BlockSpec((BN, 1), lambda i: (i, 0)),
            pl.BlockSpec((H, 2 * H), lambda i: (0, 0)),
            pl.BlockSpec((1, 2 * H), lambda i: (0, 0)),
            pl.BlockSpec((2 * H, H), lambda i: (0, 0)),
            pl.BlockSpec((1, H), lambda i: (0, 0)),
            pl.BlockSpec((H, H), lambda i: (0, 0)),
            pl.BlockSpec((1, H), lambda i: (0, 0)),
            pl.BlockSpec((H, H), lambda i: (0, 0)),
            pl.BlockSpec((1, H), lambda i: (0, 0)),
            pl.BlockSpec((H, H), lambda i: (0, 0)),
            pl.BlockSpec((1, H), lambda i: (0, 0)),
            pl.BlockSpec((BN, 2), lambda i: (i, 0)),
            pl.BlockSpec((BN, 16), lambda i: (i, 0)),
        ],
        out_specs=pl.BlockSpec((BN, 2), lambda i: (i, 0)),
        out_shape=jax.ShapeDtypeStruct((N_NODES, 2), F32),
    )(res, parts, rdeg, uw1, ub1, uw2, ub2, ow1, ob1, ow2, ob2, ow3p, ob3p, x, tab)


def kernel(h, x, vel, edges, edge_attr_orig,
           msg_W1_1, msg_b1_1, msg_W1_2, msg_b1_2, msg_W1_3, msg_b1_3,
           msg_W1_4, msg_b1_4,
           msg_W2_1, msg_b2_1, msg_W2_2, msg_b2_2, msg_W2_3, msg_b2_3,
           msg_W2_4, msg_b2_4,
           upd_W1_1, upd_b1_1, upd_W1_2, upd_b1_2, upd_W1_3, upd_b1_3,
           upd_W1_4, upd_b1_4,
           upd_W2_1, upd_b2_1, upd_W2_2, upd_b2_2, upd_W2_3, upd_b2_3,
           upd_W2_4, upd_b2_4,
           res_W_1, res_b_1, out_W1, out_b1, out_W2, out_b2, out_W3, out_b3):
    del h
    send = edges[0]
    recv = edges[1]

    msg_w1 = {2: msg_W1_2, 3: msg_W1_3, 4: msg_W1_4}
    msg_b1 = {1: msg_b1_1.reshape(1, H), 2: msg_b1_2.reshape(1, H),
              3: msg_b1_3.reshape(1, H), 4: msg_b1_4.reshape(1, H)}
    msg_w2 = {1: msg_W2_1, 2: msg_W2_2, 3: msg_W2_3, 4: msg_W2_4}
    msg_b2 = {1: msg_b2_1.reshape(1, H), 2: msg_b2_2.reshape(1, H),
              3: msg_b2_3.reshape(1, H), 4: msg_b2_4.reshape(1, H)}
    upd_w1 = {1: upd_W1_1, 2: upd_W1_2, 3: upd_W1_3, 4: upd_W1_4}
    upd_b1 = {i: b.reshape(1, 2 * H) for i, b in
              {1: upd_b1_1, 2: upd_b1_2, 3: upd_b1_3, 4: upd_b1_4}.items()}
    upd_w2 = {1: upd_W2_1, 2: upd_W2_2, 3: upd_W2_3, 4: upd_W2_4}
    upd_b2 = {i: b.reshape(1, H) for i, b in
              {1: upd_b2_1, 2: upd_b2_2, 3: upd_b2_3, 4: upd_b2_4}.items()}
    w1s = {i: msg_w1[i][0:H] for i in (2, 3, 4)}
    w1r = {i: msg_w1[i][H:2 * H] for i in (2, 3, 4)}
    w1e = {i: msg_w1[i][2 * H:3 * H] for i in (2, 3, 4)}

    BF = jnp.bfloat16
    w1_1p = jnp.concatenate([msg_W1_1, jnp.zeros((3, H), F32)], axis=0).astype(BF)
    ow3p = jnp.concatenate([out_W3, jnp.zeros((H, H - 2), F32)], axis=1)
    ob3p = jnp.concatenate([out_b3, jnp.zeros((H - 2,), F32)]).reshape(1, H)

    tab, res1 = _tc_prep(x, vel, res_W_1[2:3, :], res_b_1.reshape(1, H))

    cnt = _sc_count(recv)
    rdeg = (1.0 / jnp.maximum(cnt[0] + cnt[1], 1.0)).reshape(N_PAD, 1)

    ps, pr = _sc_gather_pair16(tab, send, recv)
    m = _tc_edge1(ps, pr, edge_attr_orig, w1_1p, msg_b1[1],
                  msg_w2[1].astype(BF), msg_b2[1])

    parts = _sc_scatter_add(m, recv)
    res = res1
    for i in (2, 3, 4):
        xn, s_tab, r_tab = _tc_node(res, parts, rdeg,
                                    upd_w1[i - 1], upd_b1[i - 1],
                                    upd_w2[i - 1], upd_b2[i - 1],
                                    w1s[i], w1r[i])
        g = _sc_gather2_add(s_tab, r_tab, send, recv)
        m = _tc_edgeN(m, g, w1e[i].astype(BF), msg_b1[i],
                      msg_w2[i].astype(BF), msg_b2[i])
        parts = _sc_scatter_add(m, recv)
        res = xn

    return _tc_final(res, parts, rdeg,
                     upd_w1[4], upd_b1[4], upd_w2[4], upd_b2[4],
                     out_W1, out_b1.reshape(1, H), out_W2, out_b2.reshape(1, H),
                     ow3p, ob3p, x, tab)


# speedup vs baseline: 1.8063x; 1.3736x over previous
import functools

import jax
import jax.numpy as jnp
from jax import lax
from jax.experimental import pallas as pl
from jax.experimental.pallas import tpu as pltpu
from jax.experimental.pallas import tpu_sc as plsc

F32 = jnp.float32
N_NODES = 10000
N_PAD = 10240
N_EDGES = 320000
H = 128
CH = 128
N_CHUNKS = N_EDGES // CH
NW = 32
ROWS_PER_TILE = N_PAD // 16
BE = 2048
BN = 1000
PI = 3.141592653589793
TWO_PI = 6.283185307179586


def _silu(z):
    return z * (1.0 / (1.0 + jnp.exp(-z)))


def _sc_worker_id():
    return lax.axis_index("s") * 2 + lax.axis_index("c")


def _sc_gather(table, idx, d):
    mesh = plsc.VectorSubcoreMesh(core_axis_name="c", subcore_axis_name="s")
    params = None if d % 128 == 0 else pltpu.CompilerParams(use_tc_tiling_on_sc=False)

    @functools.partial(
        pl.kernel, mesh=mesh,
        out_type=jax.ShapeDtypeStruct((N_EDGES, d), F32),
        compiler_params=params,
        scratch_types=[
            pltpu.VMEM((CH,), jnp.int32),
            pltpu.VMEM((CH, d), F32),
            pltpu.SemaphoreType.DMA,
        ],
    )
    def k(table_hbm, idx_hbm, out_hbm, idx_v, rows_v, sem):
        wid = _sc_worker_id()

        def body(i, carry):
            c = wid + NW * i

            @pl.when(c < N_CHUNKS)
            def _():
                base = c * CH
                pltpu.sync_copy(idx_hbm.at[pl.ds(base, CH)], idx_v)
                pltpu.async_copy(table_hbm.at[idx_v], rows_v, sem).wait()
                pltpu.sync_copy(rows_v, out_hbm.at[pl.ds(base, CH)])
            return carry

        lax.fori_loop(0, (N_CHUNKS + NW - 1) // NW, body, 0)

    return k(table, idx)


def _sc_gather2_add(s_tab, r_tab, send, recv):
    mesh = plsc.VectorSubcoreMesh(core_axis_name="c", subcore_axis_name="s")

    @functools.partial(
        pl.kernel, mesh=mesh,
        out_type=jax.ShapeDtypeStruct((N_EDGES, H), F32),
        scratch_types=[
            pltpu.VMEM((2, CH), jnp.int32),
            pltpu.VMEM((2, CH), jnp.int32),
            pltpu.VMEM((CH, H), F32),
            pltpu.VMEM((CH, H), F32),
            pltpu.VMEM((CH, H), F32),
            pltpu.VMEM((CH, H), F32),
            pltpu.SemaphoreType.DMA,
            pltpu.SemaphoreType.DMA,
        ],
    )
    def k(s_hbm, r_hbm, send_hbm, recv_hbm, out_hbm, sidx, ridx,
          rs0, rs1, rr0, rr1, sem0, sem1):
        wid = _sc_worker_id()
        rs = (rs0, rs1)
        rr = (rr0, rr1)
        sems = (sem0, sem1)

        def fetch(i, u):
            c = wid + NW * i

            @pl.when(c < N_CHUNKS)
            def _():
                base = c * CH
                pltpu.sync_copy(send_hbm.at[pl.ds(base, CH)], sidx.at[u])
                pltpu.sync_copy(recv_hbm.at[pl.ds(base, CH)], ridx.at[u])
                pltpu.async_copy(s_hbm.at[sidx.at[u]], rs[u], sems[u])
                pltpu.async_copy(r_hbm.at[ridx.at[u]], rr[u], sems[u])

        def process(i, u):
            c = wid + NW * i

            @pl.when(c < N_CHUNKS)
            def _():
                pltpu.make_async_copy(s_hbm.at[sidx.at[u]], rs[u], sems[u]).wait()
                pltpu.make_async_copy(r_hbm.at[ridx.at[u]], rr[u], sems[u]).wait()

                def addrow(r, cc):
                    for k2 in range(H // 16):
                        sl = pl.ds(k2 * 16, 16)
                        rs[u][r, sl] = rs[u][r, sl] + rr[u][r, sl]
                    return cc

                lax.fori_loop(0, CH, addrow, 0)
                pltpu.sync_copy(rs[u], out_hbm.at[pl.ds(c * CH, CH)])

        fetch(0, 0)

        def body(t, carry):
            i0 = 2 * t
            fetch(i0 + 1, 1)
            process(i0, 0)
            fetch(i0 + 2, 0)
            process(i0 + 1, 1)
            return carry

        n_steps = (N_CHUNKS + NW - 1) // NW
        lax.fori_loop(0, (n_steps + 1) // 2, body, 0)

    return k(s_tab, r_tab, send, recv)


def _sc_gather_pair16(table, send, recv):
    mesh = plsc.VectorSubcoreMesh(core_axis_name="c", subcore_axis_name="s")
    params = pltpu.CompilerParams(use_tc_tiling_on_sc=False)

    @functools.partial(
        pl.kernel, mesh=mesh,
        out_type=[jax.ShapeDtypeStruct((N_EDGES, 16), F32),
                  jax.ShapeDtypeStruct((N_EDGES, 16), F32)],
        compiler_params=params,
        scratch_types=[
            pltpu.VMEM((CH,), jnp.int32),
            pltpu.VMEM((CH,), jnp.int32),
            pltpu.VMEM((CH, 16), F32),
            pltpu.VMEM((CH, 16), F32),
            pltpu.SemaphoreType.DMA,
            pltpu.SemaphoreType.DMA,
        ],
    )
    def k(table_hbm, send_hbm, recv_hbm, ps_hbm, pr_hbm, sidx_v, ridx_v,
          rs_v, rr_v, sem_s, sem_r):
        wid = _sc_worker_id()

        def body(i, carry):
            c = wid + NW * i

            @pl.when(c < N_CHUNKS)
            def _():
                base = c * CH
                pltpu.sync_copy(send_hbm.at[pl.ds(base, CH)], sidx_v)
                pltpu.sync_copy(recv_hbm.at[pl.ds(base, CH)], ridx_v)
                cp_s = pltpu.async_copy(table_hbm.at[sidx_v], rs_v, sem_s)
                cp_r = pltpu.async_copy(table_hbm.at[ridx_v], rr_v, sem_r)
                cp_s.wait()
                cp_r.wait()
                pltpu.sync_copy(rs_v, ps_hbm.at[pl.ds(base, CH)])
                pltpu.sync_copy(rr_v, pr_hbm.at[pl.ds(base, CH)])
            return carry

        lax.fori_loop(0, (N_CHUNKS + NW - 1) // NW, body, 0)

    return k(table, send, recv)


def _sc_scatter_add(m, idx):
    mesh = plsc.VectorSubcoreMesh(core_axis_name="c", subcore_axis_name="s")

    @functools.partial(
        pl.kernel, mesh=mesh,
        out_type=jax.ShapeDtypeStruct((2, N_PAD, H), F32),
        scratch_types=[
            pltpu.VMEM((CH,), jnp.int32),
            pltpu.VMEM((CH, H), F32),
            pltpu.VMEM_SHARED((N_PAD, H), F32),
            pltpu.SemaphoreType.DMA,
        ],
    )
    def k(m_hbm, idx_hbm, out_hbm, idx_v, rows_v, acc_sh, sem):
        cid = lax.axis_index("c")
        sid = lax.axis_index("s")
        wid = sid * 2 + cid

        def zrow(r, carry):
            for k8 in range(H // 16):
                rows_v[r, pl.ds(k8 * 16, 16)] = jnp.zeros((16,), F32)
            return carry

        lax.fori_loop(0, CH, zrow, 0)
        tile_base = sid * ROWS_PER_TILE
        for j in range(ROWS_PER_TILE // CH):
            pltpu.sync_copy(rows_v, acc_sh.at[pl.ds(tile_base + j * CH, CH)])
        plsc.subcore_barrier()

        def body(i, carry):
            c = wid + NW * i

            @pl.when(c < N_CHUNKS)
            def _():
                base = c * CH
                pltpu.sync_copy(idx_hbm.at[pl.ds(base, CH)], idx_v)
                pltpu.sync_copy(m_hbm.at[pl.ds(base, CH)], rows_v)
                pltpu.sync_copy(rows_v, acc_sh.at[idx_v], add=True)
            return carry

        lax.fori_loop(0, (N_CHUNKS + NW - 1) // NW, body, 0)
        plsc.subcore_barrier()

        for j in range(ROWS_PER_TILE // CH):
            base = tile_base + j * CH
            pltpu.sync_copy(acc_sh.at[pl.ds(base, CH)], rows_v)
            pltpu.sync_copy(rows_v, out_hbm.at[cid, pl.ds(base, CH)])

    return k(m, idx)


def _sc_count(idx):
    mesh = plsc.VectorSubcoreMesh(core_axis_name="c", subcore_axis_name="s")

    @functools.partial(
        pl.kernel, mesh=mesh,
        out_type=jax.ShapeDtypeStruct((2, N_PAD), F32),
        scratch_types=[
            pltpu.VMEM((CH,), jnp.int32),
            pltpu.VMEM((CH,), F32),
            pltpu.VMEM((CH,), F32),
            pltpu.VMEM_SHARED((N_PAD,), F32),
            pltpu.SemaphoreType.DMA,
        ],
    )
    def k(idx_hbm, out_hbm, idx_v, ones_v, zeros_v, acc_sh, sem):
        cid = lax.axis_index("c")
        sid = lax.axis_index("s")
        wid = sid * 2 + cid

        for k8 in range(CH // 16):
            ones_v[pl.ds(k8 * 16, 16)] = jnp.full((16,), 1.0, F32)
            zeros_v[pl.ds(k8 * 16, 16)] = jnp.zeros((16,), F32)
        tile_base = sid * ROWS_PER_TILE
        for j in range(ROWS_PER_TILE // CH):
            pltpu.sync_copy(zeros_v, acc_sh.at[pl.ds(tile_base + j * CH, CH)])
        plsc.subcore_barrier()

        def body(i, carry):
            c = wid + NW * i

            @pl.when(c < N_CHUNKS)
            def _():
                base = c * CH
                pltpu.sync_copy(idx_hbm.at[pl.ds(base, CH)], idx_v)
                pltpu.sync_copy(ones_v, acc_sh.at[idx_v], add=True)
            return carry

        lax.fori_loop(0, (N_CHUNKS + NW - 1) // NW, body, 0)
        plsc.subcore_barrier()

        for j in range(ROWS_PER_TILE // CH):
            base = tile_base + j * CH
            pltpu.sync_copy(acc_sh.at[pl.ds(base, CH)], zeros_v)
            pltpu.sync_copy(zeros_v, out_hbm.at[cid, pl.ds(base, CH)])

    return k(idx)


def _prep_body(x_ref, vel_ref, wrow_ref, bias_ref, tab_ref, res_ref):
    xx = x_ref[...]
    vv = vel_ref[...]
    vx = vv[:, 0:1]
    vy = vv[:, 1:2]
    theta = jnp.arctan2(vy, vx)
    c = jnp.cos(theta)
    s = jnp.sin(theta)
    speed = jnp.sqrt(vx * vx + vy * vy)
    z = jnp.zeros_like(vx)
    tab_ref[...] = jnp.concatenate(
        [xx[:, 0:1], xx[:, 1:2], vx, vy, theta, c, s, speed,
         z, z, z, z, z, z, z, z], axis=1)
    res_ref[...] = speed * wrow_ref[...] + bias_ref[...]


def _tc_prep(x, vel, res_row, res_bias):
    grid = N_NODES // BN
    return pl.pallas_call(
        _prep_body,
        grid=(grid,),
        in_specs=[
            pl.BlockSpec((BN, 2), lambda i: (i, 0)),
            pl.BlockSpec((BN, 2), lambda i: (i, 0)),
            pl.BlockSpec((1, H), lambda i: (0, 0)),
            pl.BlockSpec((1, H), lambda i: (0, 0)),
        ],
        out_specs=[
            pl.BlockSpec((BN, 16), lambda i: (i, 0)),
            pl.BlockSpec((BN, H), lambda i: (i, 0)),
        ],
        out_shape=[
            jax.ShapeDtypeStruct((N_NODES, 16), F32),
            jax.ShapeDtypeStruct((N_NODES, H), F32),
        ],
    )(x, vel, res_row, res_bias)


def _edge1_body(ps_ref, pr_ref, ea_ref, w1_ref, b1_ref, w2_ref, b2_ref, m_ref):
    PsT = ps_ref[...].T
    PrT = pr_ref[...].T
    EAT = ea_ref[...].T

    def row(M, r):
        return M[r:r + 1, :]

    dx = row(PsT, 0) - row(PrT, 0)
    dy = row(PsT, 1) - row(PrT, 1)
    cr = row(PrT, 5)
    sr = row(PrT, 6)
    rrx = cr * dx + sr * dy
    rry = -sr * dx + cr * dy
    d = row(PsT, 4) - row(PrT, 4)
    reul = d - jnp.where(d > PI, TWO_PI, 0.0) + jnp.where(d < -PI, TWO_PI, 0.0)
    dist = jnp.sqrt(dx * dx + dy * dy)
    sph = jnp.arctan2(rry, rrx)
    vxs = row(PsT, 2)
    vys = row(PsT, 3)
    rvx = cr * vxs + sr * vys
    rvy = -sr * vxs + cr * vys
    spr = row(PrT, 7)
    z = jnp.zeros_like(dx)
    featT = jnp.concatenate(
        [rrx, rry, reul, dist, sph, rvx, rvy, z, z, spr, z,
         row(EAT, 0), row(EAT, 1), z, z, z], axis=0)
    feat = featT.T.astype(jnp.bfloat16)
    m1 = _silu(jnp.dot(feat, w1_ref[...], preferred_element_type=F32) + b1_ref[...])
    m_ref[...] = _silu(jnp.dot(m1.astype(jnp.bfloat16), w2_ref[...],
                               preferred_element_type=F32) + b2_ref[...])


def _tc_edge1(ps, pr, ea, w1p, b1, w2, b2):
    grid = N_EDGES // BE
    return pl.pallas_call(
        _edge1_body,
        grid=(grid,),
        in_specs=[
            pl.BlockSpec((BE, 16), lambda i: (i, 0)),
            pl.BlockSpec((BE, 16), lambda i: (i, 0)),
            pl.BlockSpec((BE, 2), lambda i: (i, 0)),
            pl.BlockSpec((16, H), lambda i: (0, 0)),
            pl.BlockSpec((1, H), lambda i: (0, 0)),
            pl.BlockSpec((H, H), lambda i: (0, 0)),
            pl.BlockSpec((1, H), lambda i: (0, 0)),
        ],
        out_specs=pl.BlockSpec((BE, H), lambda i: (i, 0)),
        out_shape=jax.ShapeDtypeStruct((N_EDGES, H), F32),
    )(ps, pr, ea, w1p, b1, w2, b2)


def _edgeN_body(mp_ref, g_ref, w1_ref, b1_ref, w2_ref, b2_ref, m_ref):
    pre = (jnp.dot(mp_ref[...].astype(jnp.bfloat16), w1_ref[...],
                   preferred_element_type=F32)
           + g_ref[...] + b1_ref[...])
    m1 = _silu(pre)
    m_ref[...] = _silu(jnp.dot(m1.astype(jnp.bfloat16), w2_ref[...],
                               preferred_element_type=F32) + b2_ref[...])


def _tc_edgeN(m_prev, g, w1e, b1, w2, b2):
    grid = N_EDGES // BE
    return pl.pallas_call(
        _edgeN_body,
        grid=(grid,),
        in_specs=[
            pl.BlockSpec((BE, H), lambda i: (i, 0)),
            pl.BlockSpec((BE, H), lambda i: (i, 0)),
            pl.BlockSpec((H, H), lambda i: (0, 0)),
            pl.BlockSpec((1, H), lambda i: (0, 0)),
            pl.BlockSpec((H, H), lambda i: (0, 0)),
            pl.BlockSpec((1, H), lambda i: (0, 0)),
        ],
        out_specs=pl.BlockSpec((BE, H), lambda i: (i, 0)),
        out_shape=jax.ShapeDtypeStruct((N_EDGES, H), F32),
    )(m_prev, g, w1e, b1, w2, b2)


def _node_body(res_ref, parts_ref, rdeg_ref, uw1_ref, ub1_ref, uw2_ref, ub2_ref,
               ws_ref, wr_ref, xn_ref, s_ref, r_ref):
    aggr = (parts_ref[0] + parts_ref[1]) * rdeg_ref[...]
    xn1 = res_ref[...] + aggr
    u = _silu(jnp.dot(xn1, uw1_ref[...], preferred_element_type=F32) + ub1_ref[...])
    u = jnp.dot(u, uw2_ref[...], preferred_element_type=F32) + ub2_ref[...]
    xn = xn1 + u
    xn_ref[...] = xn
    s_ref[...] = jnp.dot(xn, ws_ref[...], preferred_element_type=F32)
    r_ref[...] = jnp.dot(xn, wr_ref[...], preferred_element_type=F32)


def _tc_node(res, parts, rdeg, uw1, ub1, uw2, ub2, ws, wr):
    grid = N_NODES // BN
    return pl.pallas_call(
        _node_body,
        grid=(grid,),
        in_specs=[
            pl.BlockSpec((BN, H), lambda i: (i, 0)),
            pl.BlockSpec((2, BN, H), lambda i: (0, i, 0)),
            pl.BlockSpec((BN, 1), lambda i: (i, 0)),
            pl.BlockSpec((H, 2 * H), lambda i: (0, 0)),
            pl.BlockSpec((1, 2 * H), lambda i: (0, 0)),
            pl.BlockSpec((2 * H, H), lambda i: (0, 0)),
            pl.BlockSpec((1, H), lambda i: (0, 0)),
            pl.BlockSpec((H, H), lambda i: (0, 0)),
            pl.BlockSpec((H, H), lambda i: (0, 0)),
        ],
        out_specs=[
            pl.BlockSpec((BN, H), lambda i: (i, 0)),
            pl.BlockSpec((BN, H), lambda i: (i, 0)),
            pl.BlockSpec((BN, H), lambda i: (i, 0)),
        ],
        out_shape=[
            jax.ShapeDtypeStruct((N_NODES, H), F32),
            jax.ShapeDtypeStruct((N_NODES, H), F32),
            jax.ShapeDtypeStruct((N_NODES, H), F32),
        ],
    )(res, parts, rdeg, uw1, ub1, uw2, ub2, ws, wr)


def _final_body(res_ref, parts_ref, rdeg_ref, uw1_ref, ub1_ref, uw2_ref, ub2_ref,
                ow1_ref, ob1_ref, ow2_ref, ob2_ref, ow3_ref, ob3_ref,
                x_ref, tab_ref, out_ref):
    aggr = (parts_ref[0] + parts_ref[1]) * rdeg_ref[...]
    xn1 = res_ref[...] + aggr
    u = _silu(jnp.dot(xn1, uw1_ref[...], preferred_element_type=F32) + ub1_ref[...])
    u = jnp.dot(u, uw2_ref[...], preferred_element_type=F32) + ub2_ref[...]
    xn = xn1 + u
    o = _silu(jnp.dot(xn, ow1_ref[...], preferred_element_type=F32) + ob1_ref[...])
    o = _silu(jnp.dot(o, ow2_ref[...], preferred_element_type=F32) + ob2_ref[...])
    pred = jnp.dot(o, ow3_ref[...], preferred_element_type=F32) + ob3_ref[...]
    p0 = pred[:, 0:1]
    p1 = pred[:, 1:2]
    c = tab_ref[:, 5:6]
    s = tab_ref[:, 6:7]
    out_ref[...] = x_ref[...] + jnp.concatenate(
        [c * p0 - s * p1, s * p0 + c * p1], axis=1)


def _tc_final(res, parts, rdeg, uw1, ub1, uw2, ub2,
              ow1, ob1, ow2, ob2, ow3p, ob3p, x, tab):
    grid = N_NODES // BN
    return pl.pallas_call(
        _final_body,
        grid=(grid,),
        in_specs=[
            pl.BlockSpec((BN, H), lambda i: (i, 0)),
            pl.BlockSpec((2, BN, H), lambda i: (0, i, 0)),
            pl.BlockSpec((BN, 1), lambda i: (i, 0)),
            pl.BlockSpec((H, 2 * H), lambda i: (0, 0)),
            pl.BlockSpec((1, 2 * H), lambda i: (0, 0)),
            pl.BlockSpec((2 * H, H), lambda i: (0, 0)),
            pl.BlockSpec((1, H), lambda i: (0, 0)),
            pl.BlockSpec((H, H), lambda i: (0, 0)),
            pl.BlockSpec((1, H), lambda i: (0, 0)),
            pl.BlockSpec((H, H), lambda i: (0, 0)),
            pl.BlockSpec((1, H), lambda i: (0, 0)),
            pl.BlockSpec((H, H), lambda i: (0, 0)),
            pl.BlockSpec((1, H), lambda i: (0, 0)),
            pl.BlockSpec((BN, 2), lambda i: (i, 0)),
            pl.BlockSpec((BN, 16), lambda i: (i, 0)),
        ],
        out_specs=pl.BlockSpec((BN, 2), lambda i: (i, 0)),
        out_shape=jax.ShapeDtypeStruct((N_NODES, 2), F32),
    )(res, parts, rdeg, uw1, ub1, uw2, ub2, ow1, ob1, ow2, ob2, ow3p, ob3p, x, tab)


def kernel(h, x, vel, edges, edge_attr_orig,
           msg_W1_1, msg_b1_1, msg_W1_2, msg_b1_2, msg_W1_3, msg_b1_3,
           msg_W1_4, msg_b1_4,
           msg_W2_1, msg_b2_1, msg_W2_2, msg_b2_2, msg_W2_3, msg_b2_3,
           msg_W2_4, msg_b2_4,
           upd_W1_1, upd_b1_1, upd_W1_2, upd_b1_2, upd_W1_3, upd_b1_3,
           upd_W1_4, upd_b1_4,
           upd_W2_1, upd_b2_1, upd_W2_2, upd_b2_2, upd_W2_3, upd_b2_3,
           upd_W2_4, upd_b2_4,
           res_W_1, res_b_1, out_W1, out_b1, out_W2, out_b2, out_W3, out_b3):
    del h
    send = edges[0]
    recv = edges[1]

    msg_w1 = {2: msg_W1_2, 3: msg_W1_3, 4: msg_W1_4}
    msg_b1 = {1: msg_b1_1.reshape(1, H), 2: msg_b1_2.reshape(1, H),
              3: msg_b1_3.reshape(1, H), 4: msg_b1_4.reshape(1, H)}
    msg_w2 = {1: msg_W2_1, 2: msg_W2_2, 3: msg_W2_3, 4: msg_W2_4}
    msg_b2 = {1: msg_b2_1.reshape(1, H), 2: msg_b2_2.reshape(1, H),
              3: msg_b2_3.reshape(1, H), 4: msg_b2_4.reshape(1, H)}
    upd_w1 = {1: upd_W1_1, 2: upd_W1_2, 3: upd_W1_3, 4: upd_W1_4}
    upd_b1 = {i: b.reshape(1, 2 * H) for i, b in
              {1: upd_b1_1, 2: upd_b1_2, 3: upd_b1_3, 4: upd_b1_4}.items()}
    upd_w2 = {1: upd_W2_1, 2: upd_W2_2, 3: upd_W2_3, 4: upd_W2_4}
    upd_b2 = {i: b.reshape(1, H) for i, b in
              {1: upd_b2_1, 2: upd_b2_2, 3: upd_b2_3, 4: upd_b2_4}.items()}
    w1s = {i: msg_w1[i][0:H] for i in (2, 3, 4)}
    w1r = {i: msg_w1[i][H:2 * H] for i in (2, 3, 4)}
    w1e = {i: msg_w1[i][2 * H:3 * H] for i in (2, 3, 4)}

    BF = jnp.bfloat16
    w1_1p = jnp.concatenate([msg_W1_1, jnp.zeros((3, H), F32)], axis=0).astype(BF)
    ow3p = jnp.concatenate([out_W3, jnp.zeros((H, H - 2), F32)], axis=1)
    ob3p = jnp.concatenate([out_b3, jnp.zeros((H - 2,), F32)]).reshape(1, H)

    tab, res1 = _tc_prep(x, vel, res_W_1[2:3, :], res_b_1.reshape(1, H))

    cnt = _sc_count(recv)
    rdeg = (1.0 / jnp.maximum(cnt[0] + cnt[1], 1.0)).reshape(N_PAD, 1)

    ps, pr = _sc_gather_pair16(tab, send, recv)
    m = _tc_edge1(ps, pr, edge_attr_orig, w1_1p, msg_b1[1],
                  msg_w2[1].astype(BF), msg_b2[1])

    parts = _sc_scatter_add(m, recv)
    res = res1
    for i in (2, 3, 4):
        xn, s_tab, r_tab = _tc_node(res, parts, rdeg,
                                    upd_w1[i - 1], upd_b1[i - 1],
                                    upd_w2[i - 1], upd_b2[i - 1],
                                    w1s[i], w1r[i])
        g = _sc_gather2_add(s_tab, r_tab, send, recv)
        m = _tc_edgeN(m, g, w1e[i].astype(BF), msg_b1[i],
                      msg_w2[i].astype(BF), msg_b2[i])
        parts = _sc_scatter_add(m, recv)
        res = xn

    return _tc_final(res, parts, rdeg,
                     upd_w1[4], upd_b1[4], upd_w2[4], upd_b2[4],
                     out_W1, out_b1.reshape(1, H), out_W2, out_b2.reshape(1, H),
                     ow3p, ob3p, x, tab)


# speedup vs baseline: 1.9343x; 1.0708x over previous
import functools

import jax
import jax.numpy as jnp
from jax import lax
from jax.experimental import pallas as pl
from jax.experimental.pallas import tpu as pltpu
from jax.experimental.pallas import tpu_sc as plsc

F32 = jnp.float32
N_NODES = 10000
N_PAD = 10240
N_EDGES = 320000
H = 128
CH = 128
N_CHUNKS = N_EDGES // CH
NW = 32
ROWS_PER_TILE = N_PAD // 16
BE = 4000
BN = 1000
PI = 3.141592653589793
TWO_PI = 6.283185307179586


def _silu(z):
    return z * (1.0 / (1.0 + jnp.exp(-z)))


def _sc_worker_id():
    return lax.axis_index("s") * 2 + lax.axis_index("c")


def _sc_gather(table, idx, d):
    mesh = plsc.VectorSubcoreMesh(core_axis_name="c", subcore_axis_name="s")
    params = None if d % 128 == 0 else pltpu.CompilerParams(use_tc_tiling_on_sc=False)

    @functools.partial(
        pl.kernel, mesh=mesh,
        out_type=jax.ShapeDtypeStruct((N_EDGES, d), F32),
        compiler_params=params,
        scratch_types=[
            pltpu.VMEM((CH,), jnp.int32),
            pltpu.VMEM((CH, d), F32),
            pltpu.SemaphoreType.DMA,
        ],
    )
    def k(table_hbm, idx_hbm, out_hbm, idx_v, rows_v, sem):
        wid = _sc_worker_id()

        def body(i, carry):
            c = wid + NW * i

            @pl.when(c < N_CHUNKS)
            def _():
                base = c * CH
                pltpu.sync_copy(idx_hbm.at[pl.ds(base, CH)], idx_v)
                pltpu.async_copy(table_hbm.at[idx_v], rows_v, sem).wait()
                pltpu.sync_copy(rows_v, out_hbm.at[pl.ds(base, CH)])
            return carry

        lax.fori_loop(0, (N_CHUNKS + NW - 1) // NW, body, 0)

    return k(table, idx)


def _sc_gather2_add(s_tab, r_tab, send, recv):
    mesh = plsc.VectorSubcoreMesh(core_axis_name="c", subcore_axis_name="s")

    @functools.partial(
        pl.kernel, mesh=mesh,
        out_type=jax.ShapeDtypeStruct((N_EDGES, H), F32),
        scratch_types=[
            pltpu.VMEM((2, CH), jnp.int32),
            pltpu.VMEM((2, CH), jnp.int32),
            pltpu.VMEM((CH, H), F32),
            pltpu.VMEM((CH, H), F32),
            pltpu.VMEM((CH, H), F32),
            pltpu.VMEM((CH, H), F32),
            pltpu.SemaphoreType.DMA,
            pltpu.SemaphoreType.DMA,
        ],
    )
    def k(s_hbm, r_hbm, send_hbm, recv_hbm, out_hbm, sidx, ridx,
          rs0, rs1, rr0, rr1, sem0, sem1):
        wid = _sc_worker_id()
        rs = (rs0, rs1)
        rr = (rr0, rr1)
        sems = (sem0, sem1)

        def fetch(i, u):
            c = wid + NW * i

            @pl.when(c < N_CHUNKS)
            def _():
                base = c * CH
                pltpu.sync_copy(send_hbm.at[pl.ds(base, CH)], sidx.at[u])
                pltpu.sync_copy(recv_hbm.at[pl.ds(base, CH)], ridx.at[u])
                pltpu.async_copy(s_hbm.at[sidx.at[u]], rs[u], sems[u])
                pltpu.async_copy(r_hbm.at[ridx.at[u]], rr[u], sems[u])

        def process(i, u):
            c = wid + NW * i

            @pl.when(c < N_CHUNKS)
            def _():
                pltpu.make_async_copy(s_hbm.at[sidx.at[u]], rs[u], sems[u]).wait()
                pltpu.make_async_copy(r_hbm.at[ridx.at[u]], rr[u], sems[u]).wait()

                def addrow(r, cc):
                    for k2 in range(H // 16):
                        sl = pl.ds(k2 * 16, 16)
                        rs[u][r, sl] = rs[u][r, sl] + rr[u][r, sl]
                    return cc

                lax.fori_loop(0, CH, addrow, 0)
                pltpu.sync_copy(rs[u], out_hbm.at[pl.ds(c * CH, CH)])

        fetch(0, 0)

        def body(t, carry):
            i0 = 2 * t
            fetch(i0 + 1, 1)
            process(i0, 0)
            fetch(i0 + 2, 0)
            process(i0 + 1, 1)
            return carry

        n_steps = (N_CHUNKS + NW - 1) // NW
        lax.fori_loop(0, (n_steps + 1) // 2, body, 0)

    return k(s_tab, r_tab, send, recv)


def _sc_gather_pair16(table, send, recv):
    mesh = plsc.VectorSubcoreMesh(core_axis_name="c", subcore_axis_name="s")
    params = pltpu.CompilerParams(use_tc_tiling_on_sc=False)

    @functools.partial(
        pl.kernel, mesh=mesh,
        out_type=[jax.ShapeDtypeStruct((N_EDGES, 16), F32),
                  jax.ShapeDtypeStruct((N_EDGES, 16), F32)],
        compiler_params=params,
        scratch_types=[
            pltpu.VMEM((CH,), jnp.int32),
            pltpu.VMEM((CH,), jnp.int32),
            pltpu.VMEM((CH, 16), F32),
            pltpu.VMEM((CH, 16), F32),
            pltpu.SemaphoreType.DMA,
            pltpu.SemaphoreType.DMA,
        ],
    )
    def k(table_hbm, send_hbm, recv_hbm, ps_hbm, pr_hbm, sidx_v, ridx_v,
          rs_v, rr_v, sem_s, sem_r):
        wid = _sc_worker_id()

        def body(i, carry):
            c = wid + NW * i

            @pl.when(c < N_CHUNKS)
            def _():
                base = c * CH
                pltpu.sync_copy(send_hbm.at[pl.ds(base, CH)], sidx_v)
                pltpu.sync_copy(recv_hbm.at[pl.ds(base, CH)], ridx_v)
                cp_s = pltpu.async_copy(table_hbm.at[sidx_v], rs_v, sem_s)
                cp_r = pltpu.async_copy(table_hbm.at[ridx_v], rr_v, sem_r)
                cp_s.wait()
                cp_r.wait()
                pltpu.sync_copy(rs_v, ps_hbm.at[pl.ds(base, CH)])
                pltpu.sync_copy(rr_v, pr_hbm.at[pl.ds(base, CH)])
            return carry

        lax.fori_loop(0, (N_CHUNKS + NW - 1) // NW, body, 0)

    return k(table, send, recv)


def _sc_scatter_add(m, idx):
    mesh = plsc.VectorSubcoreMesh(core_axis_name="c", subcore_axis_name="s")

    @functools.partial(
        pl.kernel, mesh=mesh,
        out_type=jax.ShapeDtypeStruct((2, N_PAD, H), F32),
        scratch_types=[
            pltpu.VMEM((CH,), jnp.int32),
            pltpu.VMEM((CH, H), F32),
            pltpu.VMEM_SHARED((N_PAD, H), F32),
            pltpu.SemaphoreType.DMA,
        ],
    )
    def k(m_hbm, idx_hbm, out_hbm, idx_v, rows_v, acc_sh, sem):
        cid = lax.axis_index("c")
        sid = lax.axis_index("s")
        wid = sid * 2 + cid

        def zrow(r, carry):
            for k8 in range(H // 16):
                rows_v[r, pl.ds(k8 * 16, 16)] = jnp.zeros((16,), F32)
            return carry

        lax.fori_loop(0, CH, zrow, 0)
        tile_base = sid * ROWS_PER_TILE
        for j in range(ROWS_PER_TILE // CH):
            pltpu.sync_copy(rows_v, acc_sh.at[pl.ds(tile_base + j * CH, CH)])
        plsc.subcore_barrier()

        def body(i, carry):
            c = wid + NW * i

            @pl.when(c < N_CHUNKS)
            def _():
                base = c * CH
                pltpu.sync_copy(idx_hbm.at[pl.ds(base, CH)], idx_v)
                pltpu.sync_copy(m_hbm.at[pl.ds(base, CH)], rows_v)
                pltpu.sync_copy(rows_v, acc_sh.at[idx_v], add=True)
            return carry

        lax.fori_loop(0, (N_CHUNKS + NW - 1) // NW, body, 0)
        plsc.subcore_barrier()

        for j in range(ROWS_PER_TILE // CH):
            base = tile_base + j * CH
            pltpu.sync_copy(acc_sh.at[pl.ds(base, CH)], rows_v)
            pltpu.sync_copy(rows_v, out_hbm.at[cid, pl.ds(base, CH)])

    return k(m, idx)


def _sc_count(idx):
    mesh = plsc.VectorSubcoreMesh(core_axis_name="c", subcore_axis_name="s")

    @functools.partial(
        pl.kernel, mesh=mesh,
        out_type=jax.ShapeDtypeStruct((2, N_PAD), F32),
        scratch_types=[
            pltpu.VMEM((CH,), jnp.int32),
            pltpu.VMEM((CH,), F32),
            pltpu.VMEM((CH,), F32),
            pltpu.VMEM_SHARED((N_PAD,), F32),
            pltpu.SemaphoreType.DMA,
        ],
    )
    def k(idx_hbm, out_hbm, idx_v, ones_v, zeros_v, acc_sh, sem):
        cid = lax.axis_index("c")
        sid = lax.axis_index("s")
        wid = sid * 2 + cid

        for k8 in range(CH // 16):
            ones_v[pl.ds(k8 * 16, 16)] = jnp.full((16,), 1.0, F32)
            zeros_v[pl.ds(k8 * 16, 16)] = jnp.zeros((16,), F32)
        tile_base = sid * ROWS_PER_TILE
        for j in range(ROWS_PER_TILE // CH):
            pltpu.sync_copy(zeros_v, acc_sh.at[pl.ds(tile_base + j * CH, CH)])
        plsc.subcore_barrier()

        def body(i, carry):
            c = wid + NW * i

            @pl.when(c < N_CHUNKS)
            def _():
                base = c * CH
                pltpu.sync_copy(idx_hbm.at[pl.ds(base, CH)], idx_v)
                pltpu.sync_copy(ones_v, acc_sh.at[idx_v], add=True)
            return carry

        lax.fori_loop(0, (N_CHUNKS + NW - 1) // NW, body, 0)
        plsc.subcore_barrier()

        for j in range(ROWS_PER_TILE // CH):
            base = tile_base + j * CH
            pltpu.sync_copy(acc_sh.at[pl.ds(base, CH)], zeros_v)
            pltpu.sync_copy(zeros_v, out_hbm.at[cid, pl.ds(base, CH)])

    return k(idx)


def _prep_body(x_ref, vel_ref, wrow_ref, bias_ref, tab_ref, res_ref):
    xx = x_ref[...]
    vv = vel_ref[...]
    vx = vv[:, 0:1]
    vy = vv[:, 1:2]
    theta = jnp.arctan2(vy, vx)
    c = jnp.cos(theta)
    s = jnp.sin(theta)
    speed = jnp.sqrt(vx * vx + vy * vy)
    z = jnp.zeros_like(vx)
    tab_ref[...] = jnp.concatenate(
        [xx[:, 0:1], xx[:, 1:2], vx, vy, theta, c, s, speed,
         z, z, z, z, z, z, z, z], axis=1)
    res_ref[...] = speed * wrow_ref[...] + bias_ref[...]


def _tc_prep(x, vel, res_row, res_bias):
    grid = N_NODES // BN
    return pl.pallas_call(
        _prep_body,
        grid=(grid,),
        in_specs=[
            pl.BlockSpec((BN, 2), lambda i: (i, 0)),
            pl.BlockSpec((BN, 2), lambda i: (i, 0)),
            pl.BlockSpec((1, H), lambda i: (0, 0)),
            pl.BlockSpec((1, H), lambda i: (0, 0)),
        ],
        out_specs=[
            pl.BlockSpec((BN, 16), lambda i: (i, 0)),
            pl.BlockSpec((BN, H), lambda i: (i, 0)),
        ],
        out_shape=[
            jax.ShapeDtypeStruct((N_NODES, 16), F32),
            jax.ShapeDtypeStruct((N_NODES, H), F32),
        ],
    )(x, vel, res_row, res_bias)


def _edge1_body(ps_ref, pr_ref, ea_ref, w1_ref, b1_ref, w2_ref, b2_ref, m_ref):
    PsT = ps_ref[...].T
    PrT = pr_ref[...].T
    EAT = ea_ref[...].T

    def row(M, r):
        return M[r:r + 1, :]

    dx = row(PsT, 0) - row(PrT, 0)
    dy = row(PsT, 1) - row(PrT, 1)
    cr = row(PrT, 5)
    sr = row(PrT, 6)
    rrx = cr * dx + sr * dy
    rry = -sr * dx + cr * dy
    d = row(PsT, 4) - row(PrT, 4)
    reul = d - jnp.where(d > PI, TWO_PI, 0.0) + jnp.where(d < -PI, TWO_PI, 0.0)
    dist = jnp.sqrt(dx * dx + dy * dy)
    sph = jnp.arctan2(rry, rrx)
    vxs = row(PsT, 2)
    vys = row(PsT, 3)
    rvx = cr * vxs + sr * vys
    rvy = -sr * vxs + cr * vys
    spr = row(PrT, 7)
    z = jnp.zeros_like(dx)
    featT = jnp.concatenate(
        [rrx, rry, reul, dist, sph, rvx, rvy, z, z, spr, z,
         row(EAT, 0), row(EAT, 1), z, z, z], axis=0)
    feat = featT.T.astype(jnp.bfloat16)
    m1 = _silu(jnp.dot(feat, w1_ref[...], preferred_element_type=F32) + b1_ref[...])
    m_ref[...] = _silu(jnp.dot(m1.astype(jnp.bfloat16), w2_ref[...],
                               preferred_element_type=F32) + b2_ref[...])


def _tc_edge1(ps, pr, ea, w1p, b1, w2, b2):
    grid = N_EDGES // BE
    return pl.pallas_call(
        _edge1_body,
        grid=(grid,),
        in_specs=[
            pl.BlockSpec((BE, 16), lambda i: (i, 0)),
            pl.BlockSpec((BE, 16), lambda i: (i, 0)),
            pl.BlockSpec((BE, 2), lambda i: (i, 0)),
            pl.BlockSpec((16, H), lambda i: (0, 0)),
            pl.BlockSpec((1, H), lambda i: (0, 0)),
            pl.BlockSpec((H, H), lambda i: (0, 0)),
            pl.BlockSpec((1, H), lambda i: (0, 0)),
        ],
        out_specs=pl.BlockSpec((BE, H), lambda i: (i, 0)),
        out_shape=jax.ShapeDtypeStruct((N_EDGES, H), F32),
    )(ps, pr, ea, w1p, b1, w2, b2)


def _edgeN_body(mp_ref, g_ref, w1_ref, b1_ref, w2_ref, b2_ref, m_ref):
    pre = (jnp.dot(mp_ref[...].astype(jnp.bfloat16), w1_ref[...],
                   preferred_element_type=F32)
           + g_ref[...] + b1_ref[...])
    m1 = _silu(pre)
    m_ref[...] = _silu(jnp.dot(m1.astype(jnp.bfloat16), w2_ref[...],
                               preferred_element_type=F32) + b2_ref[...])


def _tc_edgeN(m_prev, g, w1e, b1, w2, b2):
    grid = N_EDGES // BE
    return pl.pallas_call(
        _edgeN_body,
        grid=(grid,),
        in_specs=[
            pl.BlockSpec((BE, H), lambda i: (i, 0)),
            pl.BlockSpec((BE, H), lambda i: (i, 0)),
            pl.BlockSpec((H, H), lambda i: (0, 0)),
            pl.BlockSpec((1, H), lambda i: (0, 0)),
            pl.BlockSpec((H, H), lambda i: (0, 0)),
            pl.BlockSpec((1, H), lambda i: (0, 0)),
        ],
        out_specs=pl.BlockSpec((BE, H), lambda i: (i, 0)),
        out_shape=jax.ShapeDtypeStruct((N_EDGES, H), F32),
    )(m_prev, g, w1e, b1, w2, b2)


def _node_body(res_ref, parts_ref, rdeg_ref, uw1_ref, ub1_ref, uw2_ref, ub2_ref,
               ws_ref, wr_ref, xn_ref, s_ref, r_ref):
    aggr = (parts_ref[0] + parts_ref[1]) * rdeg_ref[...]
    xn1 = res_ref[...] + aggr
    u = _silu(jnp.dot(xn1, uw1_ref[...], preferred_element_type=F32) + ub1_ref[...])
    u = jnp.dot(u, uw2_ref[...], preferred_element_type=F32) + ub2_ref[...]
    xn = xn1 + u
    xn_ref[...] = xn
    s_ref[...] = jnp.dot(xn, ws_ref[...], preferred_element_type=F32)
    r_ref[...] = jnp.dot(xn, wr_ref[...], preferred_element_type=F32)


def _tc_node(res, parts, rdeg, uw1, ub1, uw2, ub2, ws, wr):
    grid = N_NODES // BN
    return pl.pallas_call(
        _node_body,
        grid=(grid,),
        in_specs=[
            pl.BlockSpec((BN, H), lambda i: (i, 0)),
            pl.BlockSpec((2, BN, H), lambda i: (0, i, 0)),
            pl.BlockSpec((BN, 1), lambda i: (i, 0)),
            pl.BlockSpec((H, 2 * H), lambda i: (0, 0)),
            pl.BlockSpec((1, 2 * H), lambda i: (0, 0)),
            pl.BlockSpec((2 * H, H), lambda i: (0, 0)),
            pl.BlockSpec((1, H), lambda i: (0, 0)),
            pl.BlockSpec((H, H), lambda i: (0, 0)),
            pl.BlockSpec((H, H), lambda i: (0, 0)),
        ],
        out_specs=[
            pl.BlockSpec((BN, H), lambda i: (i, 0)),
            pl.BlockSpec((BN, H), lambda i: (i, 0)),
            pl.BlockSpec((BN, H), lambda i: (i, 0)),
        ],
        out_shape=[
            jax.ShapeDtypeStruct((N_NODES, H), F32),
            jax.ShapeDtypeStruct((N_NODES, H), F32),
            jax.ShapeDtypeStruct((N_NODES, H), F32),
        ],
    )(res, parts, rdeg, uw1, ub1, uw2, ub2, ws, wr)


def _final_body(res_ref, parts_ref, rdeg_ref, uw1_ref, ub1_ref, uw2_ref, ub2_ref,
                ow1_ref, ob1_ref, ow2_ref, ob2_ref, ow3_ref, ob3_ref,
                x_ref, tab_ref, out_ref):
    aggr = (parts_ref[0] + parts_ref[1]) * rdeg_ref[...]
    xn1 = res_ref[...] + aggr
    u = _silu(jnp.dot(xn1, uw1_ref[...], preferred_element_type=F32) + ub1_ref[...])
    u = jnp.dot(u, uw2_ref[...], preferred_element_type=F32) + ub2_ref[...]
    xn = xn1 + u
    o = _silu(jnp.dot(xn, ow1_ref[...], preferred_element_type=F32) + ob1_ref[...])
    o = _silu(jnp.dot(o, ow2_ref[...], preferred_element_type=F32) + ob2_ref[...])
    pred = jnp.dot(o, ow3_ref[...], preferred_element_type=F32) + ob3_ref[...]
    p0 = pred[:, 0:1]
    p1 = pred[:, 1:2]
    c = tab_ref[:, 5:6]
    s = tab_ref[:, 6:7]
    out_ref[...] = x_ref[...] + jnp.concatenate(
        [c * p0 - s * p1, s * p0 + c * p1], axis=1)


def _tc_final(res, parts, rdeg, uw1, ub1, uw2, ub2,
              ow1, ob1, ow2, ob2, ow3p, ob3p, x, tab):
    grid = N_NODES // BN
    return pl.pallas_call(
        _final_body,
        grid=(grid,),
        in_specs=[
            pl.BlockSpec((BN, H), lambda i: (i, 0)),
            pl.BlockSpec((2, BN, H), lambda i: (0, i, 0)),
            pl.BlockSpec((BN, 1), lambda i: (i, 0)),
            pl.BlockSpec((H, 2 * H), lambda i: (0, 0)),
            pl.BlockSpec((1, 2 * H), lambda i: (0, 0)),
            pl.BlockSpec((2 * H, H), lambda i: (0, 0)),
            pl.BlockSpec((1, H), lambda i: (0, 0)),
            pl.BlockSpec((H, H), lambda i: (0, 0)),
            pl.BlockSpec((1, H), lambda i: (0, 0)),
            pl.BlockSpec((H, H), lambda i: (0, 0)),
            pl.BlockSpec((1, H), lambda i: (0, 0)),
            pl.BlockSpec((H, H), lambda i: (0, 0)),
            pl.BlockSpec((1, H), lambda i: (0, 0)),
            pl.BlockSpec((BN, 2), lambda i: (i, 0)),
            pl.BlockSpec((BN, 16), lambda i: (i, 0)),
        ],
        out_specs=pl.BlockSpec((BN, 2), lambda i: (i, 0)),
        out_shape=jax.ShapeDtypeStruct((N_NODES, 2), F32),
    )(res, parts, rdeg, uw1, ub1, uw2, ub2, ow1, ob1, ow2, ob2, ow3p, ob3p, x, tab)


def kernel(h, x, vel, edges, edge_attr_orig,
           msg_W1_1, msg_b1_1, msg_W1_2, msg_b1_2, msg_W1_3, msg_b1_3,
           msg_W1_4, msg_b1_4,
           msg_W2_1, msg_b2_1, msg_W2_2, msg_b2_2, msg_W2_3, msg_b2_3,
           msg_W2_4, msg_b2_4,
           upd_W1_1, upd_b1_1, upd_W1_2, upd_b1_2, upd_W1_3, upd_b1_3,
           upd_W1_4, upd_b1_4,
           upd_W2_1, upd_b2_1, upd_W2_2, upd_b2_2, upd_W2_3, upd_b2_3,
           upd_W2_4, upd_b2_4,
           res_W_1, res_b_1, out_W1, out_b1, out_W2, out_b2, out_W3, out_b3):
    del h
    send = edges[0]
    recv = edges[1]

    msg_w1 = {2: msg_W1_2, 3: msg_W1_3, 4: msg_W1_4}
    msg_b1 = {1: msg_b1_1.reshape(1, H), 2: msg_b1_2.reshape(1, H),
              3: msg_b1_3.reshape(1, H), 4: msg_b1_4.reshape(1, H)}
    msg_w2 = {1: msg_W2_1, 2: msg_W2_2, 3: msg_W2_3, 4: msg_W2_4}
    msg_b2 = {1: msg_b2_1.reshape(1, H), 2: msg_b2_2.reshape(1, H),
              3: msg_b2_3.reshape(1, H), 4: msg_b2_4.reshape(1, H)}
    upd_w1 = {1: upd_W1_1, 2: upd_W1_2, 3: upd_W1_3, 4: upd_W1_4}
    upd_b1 = {i: b.reshape(1, 2 * H) for i, b in
              {1: upd_b1_1, 2: upd_b1_2, 3: upd_b1_3, 4: upd_b1_4}.items()}
    upd_w2 = {1: upd_W2_1, 2: upd_W2_2, 3: upd_W2_3, 4: upd_W2_4}
    upd_b2 = {i: b.reshape(1, H) for i, b in
              {1: upd_b2_1, 2: upd_b2_2, 3: upd_b2_3, 4: upd_b2_4}.items()}
    w1s = {i: msg_w1[i][0:H] for i in (2, 3, 4)}
    w1r = {i: msg_w1[i][H:2 * H] for i in (2, 3, 4)}
    w1e = {i: msg_w1[i][2 * H:3 * H] for i in (2, 3, 4)}

    BF = jnp.bfloat16
    w1_1p = jnp.concatenate([msg_W1_1, jnp.zeros((3, H), F32)], axis=0).astype(BF)
    ow3p = jnp.concatenate([out_W3, jnp.zeros((H, H - 2), F32)], axis=1)
    ob3p = jnp.concatenate([out_b3, jnp.zeros((H - 2,), F32)]).reshape(1, H)

    tab, res1 = _tc_prep(x, vel, res_W_1[2:3, :], res_b_1.reshape(1, H))

    cnt = _sc_count(recv)
    rdeg = (1.0 / jnp.maximum(cnt[0] + cnt[1], 1.0)).reshape(N_PAD, 1)

    ps, pr = _sc_gather_pair16(tab, send, recv)
    m = _tc_edge1(ps, pr, edge_attr_orig, w1_1p, msg_b1[1],
                  msg_w2[1].astype(BF), msg_b2[1])

    parts = _sc_scatter_add(m, recv)
    res = res1
    for i in (2, 3, 4):
        xn, s_tab, r_tab = _tc_node(res, parts, rdeg,
                                    upd_w1[i - 1], upd_b1[i - 1],
                                    upd_w2[i - 1], upd_b2[i - 1],
                                    w1s[i], w1r[i])
        g = _sc_gather2_add(s_tab, r_tab, send, recv)
        m = _tc_edgeN(m, g, w1e[i].astype(BF), msg_b1[i],
                      msg_w2[i].astype(BF), msg_b2[i])
        parts = _sc_scatter_add(m, recv)
        res = xn

    return _tc_final(res, parts, rdeg,
                     upd_w1[4], upd_b1[4], upd_w2[4], upd_b2[4],
                     out_W1, out_b1.reshape(1, H), out_W2, out_b2.reshape(1, H),
                     ow3p, ob3p, x, tab)


# speedup vs baseline: 2.2036x; 1.1392x over previous
import functools

import jax
import jax.numpy as jnp
from jax import lax
from jax.experimental import pallas as pl
from jax.experimental.pallas import tpu as pltpu
from jax.experimental.pallas import tpu_sc as plsc

F32 = jnp.float32
N_NODES = 10000
N_PAD = 10240
N_EDGES = 320000
H = 128
CH = 128
N_CHUNKS = N_EDGES // CH
NW = 32
ROWS_PER_TILE = N_PAD // 16
BE = 4000
BN = 1000
PI = 3.141592653589793
TWO_PI = 6.283185307179586


def _silu(z):
    return z * (1.0 / (1.0 + jnp.exp(-z)))


def _sc_worker_id():
    return lax.axis_index("s") * 2 + lax.axis_index("c")


def _sc_gather(table, idx, d):
    mesh = plsc.VectorSubcoreMesh(core_axis_name="c", subcore_axis_name="s")
    params = None if d % 128 == 0 else pltpu.CompilerParams(use_tc_tiling_on_sc=False)

    @functools.partial(
        pl.kernel, mesh=mesh,
        out_type=jax.ShapeDtypeStruct((N_EDGES, d), F32),
        compiler_params=params,
        scratch_types=[
            pltpu.VMEM((CH,), jnp.int32),
            pltpu.VMEM((CH, d), F32),
            pltpu.SemaphoreType.DMA,
        ],
    )
    def k(table_hbm, idx_hbm, out_hbm, idx_v, rows_v, sem):
        wid = _sc_worker_id()

        def body(i, carry):
            c = wid + NW * i

            @pl.when(c < N_CHUNKS)
            def _():
                base = c * CH
                pltpu.sync_copy(idx_hbm.at[pl.ds(base, CH)], idx_v)
                pltpu.async_copy(table_hbm.at[idx_v], rows_v, sem).wait()
                pltpu.sync_copy(rows_v, out_hbm.at[pl.ds(base, CH)])
            return carry

        lax.fori_loop(0, (N_CHUNKS + NW - 1) // NW, body, 0)

    return k(table, idx)


def _sc_gather2_add(s_tab, r_tab, send, recv):
    mesh = plsc.VectorSubcoreMesh(core_axis_name="c", subcore_axis_name="s")

    @functools.partial(
        pl.kernel, mesh=mesh,
        out_type=jax.ShapeDtypeStruct((N_EDGES, H), F32),
        scratch_types=[
            pltpu.VMEM((2, CH), jnp.int32),
            pltpu.VMEM((2, CH), jnp.int32),
            pltpu.VMEM((CH, H), F32),
            pltpu.VMEM((CH, H), F32),
            pltpu.VMEM((CH, H), F32),
            pltpu.VMEM((CH, H), F32),
            pltpu.SemaphoreType.DMA,
            pltpu.SemaphoreType.DMA,
        ],
    )
    def k(s_hbm, r_hbm, send_hbm, recv_hbm, out_hbm, sidx, ridx,
          rs0, rs1, rr0, rr1, sem0, sem1):
        wid = _sc_worker_id()
        rs = (rs0, rs1)
        rr = (rr0, rr1)
        sems = (sem0, sem1)

        def fetch(i, u):
            c = wid + NW * i

            @pl.when(c < N_CHUNKS)
            def _():
                base = c * CH
                pltpu.sync_copy(send_hbm.at[pl.ds(base, CH)], sidx.at[u])
                pltpu.sync_copy(recv_hbm.at[pl.ds(base, CH)], ridx.at[u])
                pltpu.async_copy(s_hbm.at[sidx.at[u]], rs[u], sems[u])
                pltpu.async_copy(r_hbm.at[ridx.at[u]], rr[u], sems[u])

        def process(i, u):
            c = wid + NW * i

            @pl.when(c < N_CHUNKS)
            def _():
                pltpu.make_async_copy(s_hbm.at[sidx.at[u]], rs[u], sems[u]).wait()
                pltpu.make_async_copy(r_hbm.at[ridx.at[u]], rr[u], sems[u]).wait()

                def addrow(r, cc):
                    for k2 in range(H // 16):
                        sl = pl.ds(k2 * 16, 16)
                        rs[u][r, sl] = rs[u][r, sl] + rr[u][r, sl]
                    return cc

                lax.fori_loop(0, CH, addrow, 0)
                pltpu.sync_copy(rs[u], out_hbm.at[pl.ds(c * CH, CH)])

        fetch(0, 0)

        def body(t, carry):
            i0 = 2 * t
            fetch(i0 + 1, 1)
            process(i0, 0)
            fetch(i0 + 2, 0)
            process(i0 + 1, 1)
            return carry

        n_steps = (N_CHUNKS + NW - 1) // NW
        lax.fori_loop(0, (n_steps + 1) // 2, body, 0)

    return k(s_tab, r_tab, send, recv)


def _sc_gather_pair16(table, send, recv):
    mesh = plsc.VectorSubcoreMesh(core_axis_name="c", subcore_axis_name="s")
    params = pltpu.CompilerParams(use_tc_tiling_on_sc=False)

    @functools.partial(
        pl.kernel, mesh=mesh,
        out_type=[jax.ShapeDtypeStruct((N_EDGES, 16), F32),
                  jax.ShapeDtypeStruct((N_EDGES, 16), F32)],
        compiler_params=params,
        scratch_types=[
            pltpu.VMEM((CH,), jnp.int32),
            pltpu.VMEM((CH,), jnp.int32),
            pltpu.VMEM((CH, 16), F32),
            pltpu.VMEM((CH, 16), F32),
            pltpu.SemaphoreType.DMA,
            pltpu.SemaphoreType.DMA,
        ],
    )
    def k(table_hbm, send_hbm, recv_hbm, ps_hbm, pr_hbm, sidx_v, ridx_v,
          rs_v, rr_v, sem_s, sem_r):
        wid = _sc_worker_id()

        def body(i, carry):
            c = wid + NW * i

            @pl.when(c < N_CHUNKS)
            def _():
                base = c * CH
                pltpu.sync_copy(send_hbm.at[pl.ds(base, CH)], sidx_v)
                pltpu.sync_copy(recv_hbm.at[pl.ds(base, CH)], ridx_v)
                cp_s = pltpu.async_copy(table_hbm.at[sidx_v], rs_v, sem_s)
                cp_r = pltpu.async_copy(table_hbm.at[ridx_v], rr_v, sem_r)
                cp_s.wait()
                cp_r.wait()
                pltpu.sync_copy(rs_v, ps_hbm.at[pl.ds(base, CH)])
                pltpu.sync_copy(rr_v, pr_hbm.at[pl.ds(base, CH)])
            return carry

        lax.fori_loop(0, (N_CHUNKS + NW - 1) // NW, body, 0)

    return k(table, send, recv)


def _sc_scatter_add(m, idx):
    mesh = plsc.VectorSubcoreMesh(core_axis_name="c", subcore_axis_name="s")

    @functools.partial(
        pl.kernel, mesh=mesh,
        out_type=jax.ShapeDtypeStruct((2, N_PAD, H), F32),
        scratch_types=[
            pltpu.VMEM((2, CH), jnp.int32),
            pltpu.VMEM((CH, H), F32),
            pltpu.VMEM((CH, H), F32),
            pltpu.VMEM_SHARED((N_PAD, H), F32),
            pltpu.SemaphoreType.DMA,
            pltpu.SemaphoreType.DMA,
            pltpu.SemaphoreType.DMA,
            pltpu.SemaphoreType.DMA,
        ],
    )
    def k(m_hbm, idx_hbm, out_hbm, idxb, rows0, rows1, acc_sh,
          sem_m0, sem_m1, sem_s0, sem_s1):
        cid = lax.axis_index("c")
        sid = lax.axis_index("s")
        wid = sid * 2 + cid
        rows = (rows0, rows1)
        sem_m = (sem_m0, sem_m1)
        sem_s = (sem_s0, sem_s1)

        def zrow(r, carry):
            for k8 in range(H // 16):
                rows0[r, pl.ds(k8 * 16, 16)] = jnp.zeros((16,), F32)
            return carry

        lax.fori_loop(0, CH, zrow, 0)
        tile_base = sid * ROWS_PER_TILE
        for j in range(ROWS_PER_TILE // CH):
            pltpu.sync_copy(rows0, acc_sh.at[pl.ds(tile_base + j * CH, CH)])
        plsc.subcore_barrier()

        def fetch(i, u):
            c = wid + NW * i

            @pl.when(c < N_CHUNKS)
            def _():
                @pl.when(i >= 2)
                def _w():
                    pltpu.make_async_copy(rows[u], acc_sh.at[idxb.at[u]],
                                          sem_s[u]).wait()
                base = c * CH
                pltpu.sync_copy(idx_hbm.at[pl.ds(base, CH)], idxb.at[u])
                pltpu.async_copy(m_hbm.at[pl.ds(base, CH)], rows[u], sem_m[u])

        def process(i, u):
            c = wid + NW * i

            @pl.when(c < N_CHUNKS)
            def _():
                pltpu.make_async_copy(m_hbm.at[pl.ds(c * CH, CH)], rows[u],
                                      sem_m[u]).wait()
                pltpu.async_copy(rows[u], acc_sh.at[idxb.at[u]], sem_s[u],
                                 add=True)

        fetch(0, 0)

        def body(t, carry):
            i0 = 2 * t
            fetch(i0 + 1, 1)
            process(i0, 0)
            fetch(i0 + 2, 0)
            process(i0 + 1, 1)
            return carry

        n_steps = (N_CHUNKS + NW - 1) // NW
        lax.fori_loop(0, (n_steps + 1) // 2, body, 0)
        pltpu.make_async_copy(rows0, acc_sh.at[idxb.at[0]], sem_s0).wait()
        pltpu.make_async_copy(rows1, acc_sh.at[idxb.at[1]], sem_s1).wait()
        plsc.subcore_barrier()

        for j in range(ROWS_PER_TILE // CH):
            base = tile_base + j * CH
            pltpu.sync_copy(acc_sh.at[pl.ds(base, CH)], rows0)
            pltpu.sync_copy(rows0, out_hbm.at[cid, pl.ds(base, CH)])

    return k(m, idx)


def _sc_count(idx):
    mesh = plsc.VectorSubcoreMesh(core_axis_name="c", subcore_axis_name="s")

    @functools.partial(
        pl.kernel, mesh=mesh,
        out_type=jax.ShapeDtypeStruct((2, N_PAD), F32),
        scratch_types=[
            pltpu.VMEM((CH,), jnp.int32),
            pltpu.VMEM((CH,), F32),
            pltpu.VMEM((CH,), F32),
            pltpu.VMEM_SHARED((N_PAD,), F32),
            pltpu.SemaphoreType.DMA,
        ],
    )
    def k(idx_hbm, out_hbm, idx_v, ones_v, zeros_v, acc_sh, sem):
        cid = lax.axis_index("c")
        sid = lax.axis_index("s")
        wid = sid * 2 + cid

        for k8 in range(CH // 16):
            ones_v[pl.ds(k8 * 16, 16)] = jnp.full((16,), 1.0, F32)
            zeros_v[pl.ds(k8 * 16, 16)] = jnp.zeros((16,), F32)
        tile_base = sid * ROWS_PER_TILE
        for j in range(ROWS_PER_TILE // CH):
            pltpu.sync_copy(zeros_v, acc_sh.at[pl.ds(tile_base + j * CH, CH)])
        plsc.subcore_barrier()

        def body(i, carry):
            c = wid + NW * i

            @pl.when(c < N_CHUNKS)
            def _():
                base = c * CH
                pltpu.sync_copy(idx_hbm.at[pl.ds(base, CH)], idx_v)
                pltpu.sync_copy(ones_v, acc_sh.at[idx_v], add=True)
            return carry

        lax.fori_loop(0, (N_CHUNKS + NW - 1) // NW, body, 0)
        plsc.subcore_barrier()

        for j in range(ROWS_PER_TILE // CH):
            base = tile_base + j * CH
            pltpu.sync_copy(acc_sh.at[pl.ds(base, CH)], zeros_v)
            pltpu.sync_copy(zeros_v, out_hbm.at[cid, pl.ds(base, CH)])

    return k(idx)


def _prep_body(x_ref, vel_ref, wrow_ref, bias_ref, tab_ref, res_ref):
    xx = x_ref[...]
    vv = vel_ref[...]
    vx = vv[:, 0:1]
    vy = vv[:, 1:2]
    theta = jnp.arctan2(vy, vx)
    c = jnp.cos(theta)
    s = jnp.sin(theta)
    speed = jnp.sqrt(vx * vx + vy * vy)
    z = jnp.zeros_like(vx)
    tab_ref[...] = jnp.concatenate(
        [xx[:, 0:1], xx[:, 1:2], vx, vy, theta, c, s, speed,
         z, z, z, z, z, z, z, z], axis=1)
    res_ref[...] = speed * wrow_ref[...] + bias_ref[...]


def _tc_prep(x, vel, res_row, res_bias):
    grid = N_NODES // BN
    return pl.pallas_call(
        _prep_body,
        grid=(grid,),
        in_specs=[
            pl.BlockSpec((BN, 2), lambda i: (i, 0)),
            pl.BlockSpec((BN, 2), lambda i: (i, 0)),
            pl.BlockSpec((1, H), lambda i: (0, 0)),
            pl.BlockSpec((1, H), lambda i: (0, 0)),
        ],
        out_specs=[
            pl.BlockSpec((BN, 16), lambda i: (i, 0)),
            pl.BlockSpec((BN, H), lambda i: (i, 0)),
        ],
        out_shape=[
            jax.ShapeDtypeStruct((N_NODES, 16), F32),
            jax.ShapeDtypeStruct((N_NODES, H), F32),
        ],
    )(x, vel, res_row, res_bias)


def _edge1_body(ps_ref, pr_ref, ea_ref, w1_ref, b1_ref, w2_ref, b2_ref, m_ref):
    PsT = ps_ref[...].T
    PrT = pr_ref[...].T
    EAT = ea_ref[...].T

    def row(M, r):
        return M[r:r + 1, :]

    dx = row(PsT, 0) - row(PrT, 0)
    dy = row(PsT, 1) - row(PrT, 1)
    cr = row(PrT, 5)
    sr = row(PrT, 6)
    rrx = cr * dx + sr * dy
    rry = -sr * dx + cr * dy
    d = row(PsT, 4) - row(PrT, 4)
    reul = d - jnp.where(d > PI, TWO_PI, 0.0) + jnp.where(d < -PI, TWO_PI, 0.0)
    dist = jnp.sqrt(dx * dx + dy * dy)
    sph = jnp.arctan2(rry, rrx)
    vxs = row(PsT, 2)
    vys = row(PsT, 3)
    rvx = cr * vxs + sr * vys
    rvy = -sr * vxs + cr * vys
    spr = row(PrT, 7)
    z = jnp.zeros_like(dx)
    featT = jnp.concatenate(
        [rrx, rry, reul, dist, sph, rvx, rvy, z, z, spr, z,
         row(EAT, 0), row(EAT, 1), z, z, z], axis=0)
    feat = featT.T.astype(jnp.bfloat16)
    m1 = _silu(jnp.dot(feat, w1_ref[...], preferred_element_type=F32) + b1_ref[...])
    m_ref[...] = _silu(jnp.dot(m1.astype(jnp.bfloat16), w2_ref[...],
                               preferred_element_type=F32) + b2_ref[...])


def _tc_edge1(ps, pr, ea, w1p, b1, w2, b2):
    grid = N_EDGES // BE
    return pl.pallas_call(
        _edge1_body,
        grid=(grid,),
        in_specs=[
            pl.BlockSpec((BE, 16), lambda i: (i, 0)),
            pl.BlockSpec((BE, 16), lambda i: (i, 0)),
            pl.BlockSpec((BE, 2), lambda i: (i, 0)),
            pl.BlockSpec((16, H), lambda i: (0, 0)),
            pl.BlockSpec((1, H), lambda i: (0, 0)),
            pl.BlockSpec((H, H), lambda i: (0, 0)),
            pl.BlockSpec((1, H), lambda i: (0, 0)),
        ],
        out_specs=pl.BlockSpec((BE, H), lambda i: (i, 0)),
        out_shape=jax.ShapeDtypeStruct((N_EDGES, H), F32),
    )(ps, pr, ea, w1p, b1, w2, b2)


def _edgeN_body(mp_ref, g_ref, w1_ref, b1_ref, w2_ref, b2_ref, m_ref):
    pre = (jnp.dot(mp_ref[...].astype(jnp.bfloat16), w1_ref[...],
                   preferred_element_type=F32)
           + g_ref[...] + b1_ref[...])
    m1 = _silu(pre)
    m_ref[...] = _silu(jnp.dot(m1.astype(jnp.bfloat16), w2_ref[...],
                               preferred_element_type=F32) + b2_ref[...])


def _tc_edgeN(m_prev, g, w1e, b1, w2, b2):
    grid = N_EDGES // BE
    return pl.pallas_call(
        _edgeN_body,
        grid=(grid,),
        in_specs=[
            pl.BlockSpec((BE, H), lambda i: (i, 0)),
            pl.BlockSpec((BE, H), lambda i: (i, 0)),
            pl.BlockSpec((H, H), lambda i: (0, 0)),
            pl.BlockSpec((1, H), lambda i: (0, 0)),
            pl.BlockSpec((H, H), lambda i: (0, 0)),
            pl.BlockSpec((1, H), lambda i: (0, 0)),
        ],
        out_specs=pl.BlockSpec((BE, H), lambda i: (i, 0)),
        out_shape=jax.ShapeDtypeStruct((N_EDGES, H), F32),
    )(m_prev, g, w1e, b1, w2, b2)


def _node_body(res_ref, parts_ref, rdeg_ref, uw1_ref, ub1_ref, uw2_ref, ub2_ref,
               ws_ref, wr_ref, xn_ref, s_ref, r_ref):
    aggr = (parts_ref[0] + parts_ref[1]) * rdeg_ref[...]
    xn1 = res_ref[...] + aggr
    u = _silu(jnp.dot(xn1, uw1_ref[...], preferred_element_type=F32) + ub1_ref[...])
    u = jnp.dot(u, uw2_ref[...], preferred_element_type=F32) + ub2_ref[...]
    xn = xn1 + u
    xn_ref[...] = xn
    s_ref[...] = jnp.dot(xn, ws_ref[...], preferred_element_type=F32)
    r_ref[...] = jnp.dot(xn, wr_ref[...], preferred_element_type=F32)


def _tc_node(res, parts, rdeg, uw1, ub1, uw2, ub2, ws, wr):
    grid = N_NODES // BN
    return pl.pallas_call(
        _node_body,
        grid=(grid,),
        in_specs=[
            pl.BlockSpec((BN, H), lambda i: (i, 0)),
            pl.BlockSpec((2, BN, H), lambda i: (0, i, 0)),
            pl.BlockSpec((BN, 1), lambda i: (i, 0)),
            pl.BlockSpec((H, 2 * H), lambda i: (0, 0)),
            pl.BlockSpec((1, 2 * H), lambda i: (0, 0)),
            pl.BlockSpec((2 * H, H), lambda i: (0, 0)),
            pl.BlockSpec((1, H), lambda i: (0, 0)),
            pl.BlockSpec((H, H), lambda i: (0, 0)),
            pl.BlockSpec((H, H), lambda i: (0, 0)),
        ],
        out_specs=[
            pl.BlockSpec((BN, H), lambda i: (i, 0)),
            pl.BlockSpec((BN, H), lambda i: (i, 0)),
            pl.BlockSpec((BN, H), lambda i: (i, 0)),
        ],
        out_shape=[
            jax.ShapeDtypeStruct((N_NODES, H), F32),
            jax.ShapeDtypeStruct((N_NODES, H), F32),
            jax.ShapeDtypeStruct((N_NODES, H), F32),
        ],
    )(res, parts, rdeg, uw1, ub1, uw2, ub2, ws, wr)


def _final_body(res_ref, parts_ref, rdeg_ref, uw1_ref, ub1_ref, uw2_ref, ub2_ref,
                ow1_ref, ob1_ref, ow2_ref, ob2_ref, ow3_ref, ob3_ref,
                x_ref, tab_ref, out_ref):
    aggr = (parts_ref[0] + parts_ref[1]) * rdeg_ref[...]
    xn1 = res_ref[...] + aggr
    u = _silu(jnp.dot(xn1, uw1_ref[...], preferred_element_type=F32) + ub1_ref[...])
    u = jnp.dot(u, uw2_ref[...], preferred_element_type=F32) + ub2_ref[...]
    xn = xn1 + u
    o = _silu(jnp.dot(xn, ow1_ref[...], preferred_element_type=F32) + ob1_ref[...])
    o = _silu(jnp.dot(o, ow2_ref[...], preferred_element_type=F32) + ob2_ref[...])
    pred = jnp.dot(o, ow3_ref[...], preferred_element_type=F32) + ob3_ref[...]
    p0 = pred[:, 0:1]
    p1 = pred[:, 1:2]
    c = tab_ref[:, 5:6]
    s = tab_ref[:, 6:7]
    out_ref[...] = x_ref[...] + jnp.concatenate(
        [c * p0 - s * p1, s * p0 + c * p1], axis=1)


def _tc_final(res, parts, rdeg, uw1, ub1, uw2, ub2,
              ow1, ob1, ow2, ob2, ow3p, ob3p, x, tab):
    grid = N_NODES // BN
    return pl.pallas_call(
        _final_body,
        grid=(grid,),
        in_specs=[
            pl.BlockSpec((BN, H), lambda i: (i, 0)),
            pl.BlockSpec((2, BN, H), lambda i: (0, i, 0)),
            pl.BlockSpec((BN, 1), lambda i: (i, 0)),
            pl.BlockSpec((H, 2 * H), lambda i: (0, 0)),
            pl.BlockSpec((1, 2 * H), lambda i: (0, 0)),
            pl.BlockSpec((2 * H, H), lambda i: (0, 0)),
            pl.BlockSpec((1, H), lambda i: (0, 0)),
            pl.BlockSpec((H, H), lambda i: (0, 0)),
            pl.BlockSpec((1, H), lambda i: (0, 0)),
            pl.BlockSpec((H, H), lambda i: (0, 0)),
            pl.BlockSpec((1, H), lambda i: (0, 0)),
            pl.BlockSpec((H, H), lambda i: (0, 0)),
            pl.BlockSpec((1, H), lambda i: (0, 0)),
            pl.BlockSpec((BN, 2), lambda i: (i, 0)),
            pl.BlockSpec((BN, 16), lambda i: (i, 0)),
        ],
        out_specs=pl.BlockSpec((BN, 2), lambda i: (i, 0)),
        out_shape=jax.ShapeDtypeStruct((N_NODES, 2), F32),
    )(res, parts, rdeg, uw1, ub1, uw2, ub2, ow1, ob1, ow2, ob2, ow3p, ob3p, x, tab)


def kernel(h, x, vel, edges, edge_attr_orig,
           msg_W1_1, msg_b1_1, msg_W1_2, msg_b1_2, msg_W1_3, msg_b1_3,
           msg_W1_4, msg_b1_4,
           msg_W2_1, msg_b2_1, msg_W2_2, msg_b2_2, msg_W2_3, msg_b2_3,
           msg_W2_4, msg_b2_4,
           upd_W1_1, upd_b1_1, upd_W1_2, upd_b1_2, upd_W1_3, upd_b1_3,
           upd_W1_4, upd_b1_4,
           upd_W2_1, upd_b2_1, upd_W2_2, upd_b2_2, upd_W2_3, upd_b2_3,
           upd_W2_4, upd_b2_4,
           res_W_1, res_b_1, out_W1, out_b1, out_W2, out_b2, out_W3, out_b3):
    del h
    send = edges[0]
    recv = edges[1]

    msg_w1 = {2: msg_W1_2, 3: msg_W1_3, 4: msg_W1_4}
    msg_b1 = {1: msg_b1_1.reshape(1, H), 2: msg_b1_2.reshape(1, H),
              3: msg_b1_3.reshape(1, H), 4: msg_b1_4.reshape(1, H)}
    msg_w2 = {1: msg_W2_1, 2: msg_W2_2, 3: msg_W2_3, 4: msg_W2_4}
    msg_b2 = {1: msg_b2_1.reshape(1, H), 2: msg_b2_2.reshape(1, H),
              3: msg_b2_3.reshape(1, H), 4: msg_b2_4.reshape(1, H)}
    upd_w1 = {1: upd_W1_1, 2: upd_W1_2, 3: upd_W1_3, 4: upd_W1_4}
    upd_b1 = {i: b.reshape(1, 2 * H) for i, b in
              {1: upd_b1_1, 2: upd_b1_2, 3: upd_b1_3, 4: upd_b1_4}.items()}
    upd_w2 = {1: upd_W2_1, 2: upd_W2_2, 3: upd_W2_3, 4: upd_W2_4}
    upd_b2 = {i: b.reshape(1, H) for i, b in
              {1: upd_b2_1, 2: upd_b2_2, 3: upd_b2_3, 4: upd_b2_4}.items()}
    w1s = {i: msg_w1[i][0:H] for i in (2, 3, 4)}
    w1r = {i: msg_w1[i][H:2 * H] for i in (2, 3, 4)}
    w1e = {i: msg_w1[i][2 * H:3 * H] for i in (2, 3, 4)}

    BF = jnp.bfloat16
    w1_1p = jnp.concatenate([msg_W1_1, jnp.zeros((3, H), F32)], axis=0).astype(BF)
    ow3p = jnp.concatenate([out_W3, jnp.zeros((H, H - 2), F32)], axis=1)
    ob3p = jnp.concatenate([out_b3, jnp.zeros((H - 2,), F32)]).reshape(1, H)

    tab, res1 = _tc_prep(x, vel, res_W_1[2:3, :], res_b_1.reshape(1, H))

    cnt = _sc_count(recv)
    rdeg = (1.0 / jnp.maximum(cnt[0] + cnt[1], 1.0)).reshape(N_PAD, 1)

    ps, pr = _sc_gather_pair16(tab, send, recv)
    m = _tc_edge1(ps, pr, edge_attr_orig, w1_1p, msg_b1[1],
                  msg_w2[1].astype(BF), msg_b2[1])

    parts = _sc_scatter_add(m, recv)
    res = res1
    for i in (2, 3, 4):
        xn, s_tab, r_tab = _tc_node(res, parts, rdeg,
                                    upd_w1[i - 1], upd_b1[i - 1],
                                    upd_w2[i - 1], upd_b2[i - 1],
                                    w1s[i], w1r[i])
        g = _sc_gather2_add(s_tab, r_tab, send, recv)
        m = _tc_edgeN(m, g, w1e[i].astype(BF), msg_b1[i],
                      msg_w2[i].astype(BF), msg_b2[i])
        parts = _sc_scatter_add(m, recv)
        res = xn

    return _tc_final(res, parts, rdeg,
                     upd_w1[4], upd_b1[4], upd_w2[4], upd_b2[4],
                     out_W1, out_b1.reshape(1, H), out_W2, out_b2.reshape(1, H),
                     ow3p, ob3p, x, tab)


# speedup vs baseline: 2.2681x; 1.0293x over previous
import functools

import jax
import jax.numpy as jnp
from jax import lax
from jax.experimental import pallas as pl
from jax.experimental.pallas import tpu as pltpu
from jax.experimental.pallas import tpu_sc as plsc

F32 = jnp.float32
N_NODES = 10000
N_PAD = 10240
N_EDGES = 320000
H = 128
CH = 128
N_CHUNKS = N_EDGES // CH
NW = 32
ROWS_PER_TILE = N_PAD // 16
BE = 8000
BN = 1000
PI = 3.141592653589793
TWO_PI = 6.283185307179586


def _silu(z):
    return z * (1.0 / (1.0 + jnp.exp(-z)))


def _sc_worker_id():
    return lax.axis_index("s") * 2 + lax.axis_index("c")


def _sc_gather(table, idx, d):
    mesh = plsc.VectorSubcoreMesh(core_axis_name="c", subcore_axis_name="s")
    params = None if d % 128 == 0 else pltpu.CompilerParams(use_tc_tiling_on_sc=False)

    @functools.partial(
        pl.kernel, mesh=mesh,
        out_type=jax.ShapeDtypeStruct((N_EDGES, d), F32),
        compiler_params=params,
        scratch_types=[
            pltpu.VMEM((CH,), jnp.int32),
            pltpu.VMEM((CH, d), F32),
            pltpu.SemaphoreType.DMA,
        ],
    )
    def k(table_hbm, idx_hbm, out_hbm, idx_v, rows_v, sem):
        wid = _sc_worker_id()

        def body(i, carry):
            c = wid + NW * i

            @pl.when(c < N_CHUNKS)
            def _():
                base = c * CH
                pltpu.sync_copy(idx_hbm.at[pl.ds(base, CH)], idx_v)
                pltpu.async_copy(table_hbm.at[idx_v], rows_v, sem).wait()
                pltpu.sync_copy(rows_v, out_hbm.at[pl.ds(base, CH)])
            return carry

        lax.fori_loop(0, (N_CHUNKS + NW - 1) // NW, body, 0)

    return k(table, idx)


def _sc_gather2_add(s_tab, r_tab, send, recv):
    mesh = plsc.VectorSubcoreMesh(core_axis_name="c", subcore_axis_name="s")

    @functools.partial(
        pl.kernel, mesh=mesh,
        out_type=jax.ShapeDtypeStruct((N_EDGES, H), F32),
        scratch_types=[
            pltpu.VMEM((2, CH), jnp.int32),
            pltpu.VMEM((2, CH), jnp.int32),
            pltpu.VMEM((CH, H), F32),
            pltpu.VMEM((CH, H), F32),
            pltpu.VMEM((CH, H), F32),
            pltpu.VMEM((CH, H), F32),
            pltpu.SemaphoreType.DMA,
            pltpu.SemaphoreType.DMA,
        ],
    )
    def k(s_hbm, r_hbm, send_hbm, recv_hbm, out_hbm, sidx, ridx,
          rs0, rs1, rr0, rr1, sem0, sem1):
        wid = _sc_worker_id()
        rs = (rs0, rs1)
        rr = (rr0, rr1)
        sems = (sem0, sem1)

        def fetch(i, u):
            c = wid + NW * i

            @pl.when(c < N_CHUNKS)
            def _():
                base = c * CH
                pltpu.sync_copy(send_hbm.at[pl.ds(base, CH)], sidx.at[u])
                pltpu.sync_copy(recv_hbm.at[pl.ds(base, CH)], ridx.at[u])
                pltpu.async_copy(s_hbm.at[sidx.at[u]], rs[u], sems[u])
                pltpu.async_copy(r_hbm.at[ridx.at[u]], rr[u], sems[u])

        def process(i, u):
            c = wid + NW * i

            @pl.when(c < N_CHUNKS)
            def _():
                pltpu.make_async_copy(s_hbm.at[sidx.at[u]], rs[u], sems[u]).wait()
                pltpu.make_async_copy(r_hbm.at[ridx.at[u]], rr[u], sems[u]).wait()

                def addrow(r, cc):
                    for k2 in range(H // 16):
                        sl = pl.ds(k2 * 16, 16)
                        rs[u][r, sl] = rs[u][r, sl] + rr[u][r, sl]
                    return cc

                lax.fori_loop(0, CH, addrow, 0)
                pltpu.sync_copy(rs[u], out_hbm.at[pl.ds(c * CH, CH)])

        fetch(0, 0)

        def body(t, carry):
            i0 = 2 * t
            fetch(i0 + 1, 1)
            process(i0, 0)
            fetch(i0 + 2, 0)
            process(i0 + 1, 1)
            return carry

        n_steps = (N_CHUNKS + NW - 1) // NW
        lax.fori_loop(0, (n_steps + 1) // 2, body, 0)

    return k(s_tab, r_tab, send, recv)


def _sc_gather_pair16(table, send, recv):
    mesh = plsc.VectorSubcoreMesh(core_axis_name="c", subcore_axis_name="s")
    params = pltpu.CompilerParams(use_tc_tiling_on_sc=False)

    @functools.partial(
        pl.kernel, mesh=mesh,
        out_type=[jax.ShapeDtypeStruct((N_EDGES, 16), F32),
                  jax.ShapeDtypeStruct((N_EDGES, 16), F32)],
        compiler_params=params,
        scratch_types=[
            pltpu.VMEM((CH,), jnp.int32),
            pltpu.VMEM((CH,), jnp.int32),
            pltpu.VMEM((CH, 16), F32),
            pltpu.VMEM((CH, 16), F32),
            pltpu.SemaphoreType.DMA,
            pltpu.SemaphoreType.DMA,
        ],
    )
    def k(table_hbm, send_hbm, recv_hbm, ps_hbm, pr_hbm, sidx_v, ridx_v,
          rs_v, rr_v, sem_s, sem_r):
        wid = _sc_worker_id()

        def body(i, carry):
            c = wid + NW * i

            @pl.when(c < N_CHUNKS)
            def _():
                base = c * CH
                pltpu.sync_copy(send_hbm.at[pl.ds(base, CH)], sidx_v)
                pltpu.sync_copy(recv_hbm.at[pl.ds(base, CH)], ridx_v)
                cp_s = pltpu.async_copy(table_hbm.at[sidx_v], rs_v, sem_s)
                cp_r = pltpu.async_copy(table_hbm.at[ridx_v], rr_v, sem_r)
                cp_s.wait()
                cp_r.wait()
                pltpu.sync_copy(rs_v, ps_hbm.at[pl.ds(base, CH)])
                pltpu.sync_copy(rr_v, pr_hbm.at[pl.ds(base, CH)])
            return carry

        lax.fori_loop(0, (N_CHUNKS + NW - 1) // NW, body, 0)

    return k(table, send, recv)


def _sc_scatter_add(m, idx):
    mesh = plsc.VectorSubcoreMesh(core_axis_name="c", subcore_axis_name="s")

    @functools.partial(
        pl.kernel, mesh=mesh,
        out_type=jax.ShapeDtypeStruct((2, N_PAD, H), F32),
        scratch_types=[
            pltpu.VMEM((2, CH), jnp.int32),
            pltpu.VMEM((CH, H), F32),
            pltpu.VMEM((CH, H), F32),
            pltpu.VMEM_SHARED((N_PAD, H), F32),
            pltpu.SemaphoreType.DMA,
            pltpu.SemaphoreType.DMA,
            pltpu.SemaphoreType.DMA,
            pltpu.SemaphoreType.DMA,
        ],
    )
    def k(m_hbm, idx_hbm, out_hbm, idxb, rows0, rows1, acc_sh,
          sem_m0, sem_m1, sem_s0, sem_s1):
        cid = lax.axis_index("c")
        sid = lax.axis_index("s")
        wid = sid * 2 + cid
        rows = (rows0, rows1)
        sem_m = (sem_m0, sem_m1)
        sem_s = (sem_s0, sem_s1)

        def zrow(r, carry):
            for k8 in range(H // 16):
                rows0[r, pl.ds(k8 * 16, 16)] = jnp.zeros((16,), F32)
            return carry

        lax.fori_loop(0, CH, zrow, 0)
        tile_base = sid * ROWS_PER_TILE
        for j in range(ROWS_PER_TILE // CH):
            pltpu.sync_copy(rows0, acc_sh.at[pl.ds(tile_base + j * CH, CH)])
        plsc.subcore_barrier()

        def fetch(i, u):
            c = wid + NW * i

            @pl.when(c < N_CHUNKS)
            def _():
                @pl.when(i >= 2)
                def _w():
                    pltpu.make_async_copy(rows[u], acc_sh.at[idxb.at[u]],
                                          sem_s[u]).wait()
                base = c * CH
                pltpu.sync_copy(idx_hbm.at[pl.ds(base, CH)], idxb.at[u])
                pltpu.async_copy(m_hbm.at[pl.ds(base, CH)], rows[u], sem_m[u])

        def process(i, u):
            c = wid + NW * i

            @pl.when(c < N_CHUNKS)
            def _():
                pltpu.make_async_copy(m_hbm.at[pl.ds(c * CH, CH)], rows[u],
                                      sem_m[u]).wait()
                pltpu.async_copy(rows[u], acc_sh.at[idxb.at[u]], sem_s[u],
                                 add=True)

        fetch(0, 0)

        def body(t, carry):
            i0 = 2 * t
            fetch(i0 + 1, 1)
            process(i0, 0)
            fetch(i0 + 2, 0)
            process(i0 + 1, 1)
            return carry

        n_steps = (N_CHUNKS + NW - 1) // NW
        lax.fori_loop(0, (n_steps + 1) // 2, body, 0)
        pltpu.make_async_copy(rows0, acc_sh.at[idxb.at[0]], sem_s0).wait()
        pltpu.make_async_copy(rows1, acc_sh.at[idxb.at[1]], sem_s1).wait()
        plsc.subcore_barrier()

        for j in range(ROWS_PER_TILE // CH):
            base = tile_base + j * CH
            pltpu.sync_copy(acc_sh.at[pl.ds(base, CH)], rows0)
            pltpu.sync_copy(rows0, out_hbm.at[cid, pl.ds(base, CH)])

    return k(m, idx)


def _sc_count(idx):
    mesh = plsc.VectorSubcoreMesh(core_axis_name="c", subcore_axis_name="s")

    @functools.partial(
        pl.kernel, mesh=mesh,
        out_type=jax.ShapeDtypeStruct((2, N_PAD), F32),
        scratch_types=[
            pltpu.VMEM((CH,), jnp.int32),
            pltpu.VMEM((CH,), F32),
            pltpu.VMEM((CH,), F32),
            pltpu.VMEM_SHARED((N_PAD,), F32),
            pltpu.SemaphoreType.DMA,
        ],
    )
    def k(idx_hbm, out_hbm, idx_v, ones_v, zeros_v, acc_sh, sem):
        cid = lax.axis_index("c")
        sid = lax.axis_index("s")
        wid = sid * 2 + cid

        for k8 in range(CH // 16):
            ones_v[pl.ds(k8 * 16, 16)] = jnp.full((16,), 1.0, F32)
            zeros_v[pl.ds(k8 * 16, 16)] = jnp.zeros((16,), F32)
        tile_base = sid * ROWS_PER_TILE
        for j in range(ROWS_PER_TILE // CH):
            pltpu.sync_copy(zeros_v, acc_sh.at[pl.ds(tile_base + j * CH, CH)])
        plsc.subcore_barrier()

        def body(i, carry):
            c = wid + NW * i

            @pl.when(c < N_CHUNKS)
            def _():
                base = c * CH
                pltpu.sync_copy(idx_hbm.at[pl.ds(base, CH)], idx_v)
                pltpu.sync_copy(ones_v, acc_sh.at[idx_v], add=True)
            return carry

        lax.fori_loop(0, (N_CHUNKS + NW - 1) // NW, body, 0)
        plsc.subcore_barrier()

        for j in range(ROWS_PER_TILE // CH):
            base = tile_base + j * CH
            pltpu.sync_copy(acc_sh.at[pl.ds(base, CH)], zeros_v)
            pltpu.sync_copy(zeros_v, out_hbm.at[cid, pl.ds(base, CH)])

    return k(idx)


def _prep_body(x_ref, vel_ref, wrow_ref, bias_ref, tab_ref, res_ref):
    xx = x_ref[...]
    vv = vel_ref[...]
    vx = vv[:, 0:1]
    vy = vv[:, 1:2]
    theta = jnp.arctan2(vy, vx)
    c = jnp.cos(theta)
    s = jnp.sin(theta)
    speed = jnp.sqrt(vx * vx + vy * vy)
    z = jnp.zeros_like(vx)
    tab_ref[...] = jnp.concatenate(
        [xx[:, 0:1], xx[:, 1:2], vx, vy, theta, c, s, speed,
         z, z, z, z, z, z, z, z], axis=1)
    res_ref[...] = speed * wrow_ref[...] + bias_ref[...]


def _tc_prep(x, vel, res_row, res_bias):
    grid = N_NODES // BN
    return pl.pallas_call(
        _prep_body,
        grid=(grid,),
        in_specs=[
            pl.BlockSpec((BN, 2), lambda i: (i, 0)),
            pl.BlockSpec((BN, 2), lambda i: (i, 0)),
            pl.BlockSpec((1, H), lambda i: (0, 0)),
            pl.BlockSpec((1, H), lambda i: (0, 0)),
        ],
        out_specs=[
            pl.BlockSpec((BN, 16), lambda i: (i, 0)),
            pl.BlockSpec((BN, H), lambda i: (i, 0)),
        ],
        out_shape=[
            jax.ShapeDtypeStruct((N_NODES, 16), F32),
            jax.ShapeDtypeStruct((N_NODES, H), F32),
        ],
    )(x, vel, res_row, res_bias)


def _edge1_body(ps_ref, pr_ref, ea_ref, w1_ref, b1_ref, w2_ref, b2_ref, m_ref):
    PsT = ps_ref[...].T
    PrT = pr_ref[...].T
    EAT = ea_ref[...].T

    def row(M, r):
        return M[r:r + 1, :]

    dx = row(PsT, 0) - row(PrT, 0)
    dy = row(PsT, 1) - row(PrT, 1)
    cr = row(PrT, 5)
    sr = row(PrT, 6)
    rrx = cr * dx + sr * dy
    rry = -sr * dx + cr * dy
    d = row(PsT, 4) - row(PrT, 4)
    reul = d - jnp.where(d > PI, TWO_PI, 0.0) + jnp.where(d < -PI, TWO_PI, 0.0)
    dist = jnp.sqrt(dx * dx + dy * dy)
    sph = jnp.arctan2(rry, rrx)
    vxs = row(PsT, 2)
    vys = row(PsT, 3)
    rvx = cr * vxs + sr * vys
    rvy = -sr * vxs + cr * vys
    spr = row(PrT, 7)
    z = jnp.zeros_like(dx)
    featT = jnp.concatenate(
        [rrx, rry, reul, dist, sph, rvx, rvy, z, z, spr, z,
         row(EAT, 0), row(EAT, 1), z, z, z], axis=0)
    feat = featT.T.astype(jnp.bfloat16)
    m1 = _silu(jnp.dot(feat, w1_ref[...], preferred_element_type=F32) + b1_ref[...])
    m_ref[...] = _silu(jnp.dot(m1.astype(jnp.bfloat16), w2_ref[...],
                               preferred_element_type=F32) + b2_ref[...])


def _tc_edge1(ps, pr, ea, w1p, b1, w2, b2):
    grid = N_EDGES // BE
    return pl.pallas_call(
        _edge1_body,
        grid=(grid,),
        in_specs=[
            pl.BlockSpec((BE, 16), lambda i: (i, 0)),
            pl.BlockSpec((BE, 16), lambda i: (i, 0)),
            pl.BlockSpec((BE, 2), lambda i: (i, 0)),
            pl.BlockSpec((16, H), lambda i: (0, 0)),
            pl.BlockSpec((1, H), lambda i: (0, 0)),
            pl.BlockSpec((H, H), lambda i: (0, 0)),
            pl.BlockSpec((1, H), lambda i: (0, 0)),
        ],
        out_specs=pl.BlockSpec((BE, H), lambda i: (i, 0)),
        out_shape=jax.ShapeDtypeStruct((N_EDGES, H), F32),
    )(ps, pr, ea, w1p, b1, w2, b2)


def _edgeN_body(mp_ref, g_ref, w1_ref, b1_ref, w2_ref, b2_ref, m_ref):
    pre = (jnp.dot(mp_ref[...].astype(jnp.bfloat16), w1_ref[...],
                   preferred_element_type=F32)
           + g_ref[...] + b1_ref[...])
    m1 = _silu(pre)
    m_ref[...] = _silu(jnp.dot(m1.astype(jnp.bfloat16), w2_ref[...],
                               preferred_element_type=F32) + b2_ref[...])


def _tc_edgeN(m_prev, g, w1e, b1, w2, b2):
    grid = N_EDGES // BE
    return pl.pallas_call(
        _edgeN_body,
        grid=(grid,),
        in_specs=[
            pl.BlockSpec((BE, H), lambda i: (i, 0)),
            pl.BlockSpec((BE, H), lambda i: (i, 0)),
            pl.BlockSpec((H, H), lambda i: (0, 0)),
            pl.BlockSpec((1, H), lambda i: (0, 0)),
            pl.BlockSpec((H, H), lambda i: (0, 0)),
            pl.BlockSpec((1, H), lambda i: (0, 0)),
        ],
        out_specs=pl.BlockSpec((BE, H), lambda i: (i, 0)),
        out_shape=jax.ShapeDtypeStruct((N_EDGES, H), F32),
    )(m_prev, g, w1e, b1, w2, b2)


def _node_body(res_ref, parts_ref, rdeg_ref, uw1_ref, ub1_ref, uw2_ref, ub2_ref,
               ws_ref, wr_ref, xn_ref, s_ref, r_ref):
    aggr = (parts_ref[0] + parts_ref[1]) * rdeg_ref[...]
    xn1 = res_ref[...] + aggr
    u = _silu(jnp.dot(xn1, uw1_ref[...], preferred_element_type=F32) + ub1_ref[...])
    u = jnp.dot(u, uw2_ref[...], preferred_element_type=F32) + ub2_ref[...]
    xn = xn1 + u
    xn_ref[...] = xn
    s_ref[...] = jnp.dot(xn, ws_ref[...], preferred_element_type=F32)
    r_ref[...] = jnp.dot(xn, wr_ref[...], preferred_element_type=F32)


def _tc_node(res, parts, rdeg, uw1, ub1, uw2, ub2, ws, wr):
    grid = N_NODES // BN
    return pl.pallas_call(
        _node_body,
        grid=(grid,),
        in_specs=[
            pl.BlockSpec((BN, H), lambda i: (i, 0)),
            pl.BlockSpec((2, BN, H), lambda i: (0, i, 0)),
            pl.BlockSpec((BN, 1), lambda i: (i, 0)),
            pl.BlockSpec((H, 2 * H), lambda i: (0, 0)),
            pl.BlockSpec((1, 2 * H), lambda i: (0, 0)),
            pl.BlockSpec((2 * H, H), lambda i: (0, 0)),
            pl.BlockSpec((1, H), lambda i: (0, 0)),
            pl.BlockSpec((H, H), lambda i: (0, 0)),
            pl.BlockSpec((H, H), lambda i: (0, 0)),
        ],
        out_specs=[
            pl.BlockSpec((BN, H), lambda i: (i, 0)),
            pl.BlockSpec((BN, H), lambda i: (i, 0)),
            pl.BlockSpec((BN, H), lambda i: (i, 0)),
        ],
        out_shape=[
            jax.ShapeDtypeStruct((N_NODES, H), F32),
            jax.ShapeDtypeStruct((N_NODES, H), F32),
            jax.ShapeDtypeStruct((N_NODES, H), F32),
        ],
    )(res, parts, rdeg, uw1, ub1, uw2, ub2, ws, wr)


def _final_body(res_ref, parts_ref, rdeg_ref, uw1_ref, ub1_ref, uw2_ref, ub2_ref,
                ow1_ref, ob1_ref, ow2_ref, ob2_ref, ow3_ref, ob3_ref,
                x_ref, tab_ref, out_ref):
    aggr = (parts_ref[0] + parts_ref[1]) * rdeg_ref[...]
    xn1 = res_ref[...] + aggr
    u = _silu(jnp.dot(xn1, uw1_ref[...], preferred_element_type=F32) + ub1_ref[...])
    u = jnp.dot(u, uw2_ref[...], preferred_element_type=F32) + ub2_ref[...]
    xn = xn1 + u
    o = _silu(jnp.dot(xn, ow1_ref[...], preferred_element_type=F32) + ob1_ref[...])
    o = _silu(jnp.dot(o, ow2_ref[...], preferred_element_type=F32) + ob2_ref[...])
    pred = jnp.dot(o, ow3_ref[...], preferred_element_type=F32) + ob3_ref[...]
    p0 = pred[:, 0:1]
    p1 = pred[:, 1:2]
    c = tab_ref[:, 5:6]
    s = tab_ref[:, 6:7]
    out_ref[...] = x_ref[...] + jnp.concatenate(
        [c * p0 - s * p1, s * p0 + c * p1], axis=1)


def _tc_final(res, parts, rdeg, uw1, ub1, uw2, ub2,
              ow1, ob1, ow2, ob2, ow3p, ob3p, x, tab):
    grid = N_NODES // BN
    return pl.pallas_call(
        _final_body,
        grid=(grid,),
        in_specs=[
            pl.BlockSpec((BN, H), lambda i: (i, 0)),
            pl.BlockSpec((2, BN, H), lambda i: (0, i, 0)),
            pl.BlockSpec((BN, 1), lambda i: (i, 0)),
            pl.BlockSpec((H, 2 * H), lambda i: (0, 0)),
            pl.BlockSpec((1, 2 * H), lambda i: (0, 0)),
            pl.BlockSpec((2 * H, H), lambda i: (0, 0)),
            pl.BlockSpec((1, H), lambda i: (0, 0)),
            pl.BlockSpec((H, H), lambda i: (0, 0)),
            pl.BlockSpec((1, H), lambda i: (0, 0)),
            pl.BlockSpec((H, H), lambda i: (0, 0)),
            pl.BlockSpec((1, H), lambda i: (0, 0)),
            pl.BlockSpec((H, H), lambda i: (0, 0)),
            pl.BlockSpec((1, H), lambda i: (0, 0)),
            pl.BlockSpec((BN, 2), lambda i: (i, 0)),
            pl.BlockSpec((BN, 16), lambda i: (i, 0)),
        ],
        out_specs=pl.BlockSpec((BN, 2), lambda i: (i, 0)),
        out_shape=jax.ShapeDtypeStruct((N_NODES, 2), F32),
    )(res, parts, rdeg, uw1, ub1, uw2, ub2, ow1, ob1, ow2, ob2, ow3p, ob3p, x, tab)


def kernel(h, x, vel, edges, edge_attr_orig,
           msg_W1_1, msg_b1_1, msg_W1_2, msg_b1_2, msg_W1_3, msg_b1_3,
           msg_W1_4, msg_b1_4,
           msg_W2_1, msg_b2_1, msg_W2_2, msg_b2_2, msg_W2_3, msg_b2_3,
           msg_W2_4, msg_b2_4,
           upd_W1_1, upd_b1_1, upd_W1_2, upd_b1_2, upd_W1_3, upd_b1_3,
           upd_W1_4, upd_b1_4,
           upd_W2_1, upd_b2_1, upd_W2_2, upd_b2_2, upd_W2_3, upd_b2_3,
           upd_W2_4, upd_b2_4,
           res_W_1, res_b_1, out_W1, out_b1, out_W2, out_b2, out_W3, out_b3):
    del h
    send = edges[0]
    recv = edges[1]

    msg_w1 = {2: msg_W1_2, 3: msg_W1_3, 4: msg_W1_4}
    msg_b1 = {1: msg_b1_1.reshape(1, H), 2: msg_b1_2.reshape(1, H),
              3: msg_b1_3.reshape(1, H), 4: msg_b1_4.reshape(1, H)}
    msg_w2 = {1: msg_W2_1, 2: msg_W2_2, 3: msg_W2_3, 4: msg_W2_4}
    msg_b2 = {1: msg_b2_1.reshape(1, H), 2: msg_b2_2.reshape(1, H),
              3: msg_b2_3.reshape(1, H), 4: msg_b2_4.reshape(1, H)}
    upd_w1 = {1: upd_W1_1, 2: upd_W1_2, 3: upd_W1_3, 4: upd_W1_4}
    upd_b1 = {i: b.reshape(1, 2 * H) for i, b in
              {1: upd_b1_1, 2: upd_b1_2, 3: upd_b1_3, 4: upd_b1_4}.items()}
    upd_w2 = {1: upd_W2_1, 2: upd_W2_2, 3: upd_W2_3, 4: upd_W2_4}
    upd_b2 = {i: b.reshape(1, H) for i, b in
              {1: upd_b2_1, 2: upd_b2_2, 3: upd_b2_3, 4: upd_b2_4}.items()}
    w1s = {i: msg_w1[i][0:H] for i in (2, 3, 4)}
    w1r = {i: msg_w1[i][H:2 * H] for i in (2, 3, 4)}
    w1e = {i: msg_w1[i][2 * H:3 * H] for i in (2, 3, 4)}

    BF = jnp.bfloat16
    w1_1p = jnp.concatenate([msg_W1_1, jnp.zeros((3, H), F32)], axis=0).astype(BF)
    ow3p = jnp.concatenate([out_W3, jnp.zeros((H, H - 2), F32)], axis=1)
    ob3p = jnp.concatenate([out_b3, jnp.zeros((H - 2,), F32)]).reshape(1, H)

    tab, res1 = _tc_prep(x, vel, res_W_1[2:3, :], res_b_1.reshape(1, H))

    cnt = _sc_count(recv)
    rdeg = (1.0 / jnp.maximum(cnt[0] + cnt[1], 1.0)).reshape(N_PAD, 1)

    ps, pr = _sc_gather_pair16(tab, send, recv)
    m = _tc_edge1(ps, pr, edge_attr_orig, w1_1p, msg_b1[1],
                  msg_w2[1].astype(BF), msg_b2[1])

    parts = _sc_scatter_add(m, recv)
    res = res1
    for i in (2, 3, 4):
        xn, s_tab, r_tab = _tc_node(res, parts, rdeg,
                                    upd_w1[i - 1], upd_b1[i - 1],
                                    upd_w2[i - 1], upd_b2[i - 1],
                                    w1s[i], w1r[i])
        g = _sc_gather2_add(s_tab, r_tab, send, recv)
        m = _tc_edgeN(m, g, w1e[i].astype(BF), msg_b1[i],
                      msg_w2[i].astype(BF), msg_b2[i])
        parts = _sc_scatter_add(m, recv)
        res = xn

    return _tc_final(res, parts, rdeg,
                     upd_w1[4], upd_b1[4], upd_w2[4], upd_b2[4],
                     out_W1, out_b1.reshape(1, H), out_W2, out_b2.reshape(1, H),
                     ow3p, ob3p, x, tab)
